# baseline (speedup 1.0000x reference)
import math

import jax
import jax.numpy as jnp
from jax import lax
from jax.experimental import pallas as pl
from jax.experimental.pallas import tpu as pltpu


def _bn_scale_bias(gamma, beta, mean, var, eps=1e-5):
    s = gamma / jnp.sqrt(var + eps)
    return s, beta - mean * s


def _make_conv_body(th, W, cl, cg, ocl):
    C = cl + cg
    bf = jnp.bfloat16

    def body(xl_ref, xg_ref, tl_ref, tg_ref, bl_ref, bg_ref, w_ref,
             outl_ref, l2g_ref, xp_ref, col_ref):
        i = pl.program_id(1)
        n = pl.num_programs(1)

        xp_ref[1:th + 1, 1:W + 1, :cl] = xl_ref[0].astype(bf)
        xp_ref[1:th + 1, 1:W + 1, cl:] = xg_ref[0].astype(bf)

        @pl.when(i == 0)
        def _():
            xp_ref[0:1, 1:W + 1, :cl] = xl_ref[0, 1:2].astype(bf)
            xp_ref[0:1, 1:W + 1, cl:] = xg_ref[0, 1:2].astype(bf)

        @pl.when(i > 0)
        def _():
            xp_ref[0:1, 1:W + 1, :cl] = tl_ref[0, 7:8].astype(bf)
            xp_ref[0:1, 1:W + 1, cl:] = tg_ref[0, 7:8].astype(bf)

        @pl.when(i == n - 1)
        def _():
            xp_ref[th + 1:th + 2, 1:W + 1, :cl] = xl_ref[0, th - 2:th - 1].astype(bf)
            xp_ref[th + 1:th + 2, 1:W + 1, cl:] = xg_ref[0, th - 2:th - 1].astype(bf)

        @pl.when(i < n - 1)
        def _():
            xp_ref[th + 1:th + 2, 1:W + 1, :cl] = bl_ref[0, 0:1].astype(bf)
            xp_ref[th + 1:th + 2, 1:W + 1, cl:] = bg_ref[0, 0:1].astype(bf)

        xp_ref[:, 0:1, :] = xp_ref[:, 2:3, :]
        xp_ref[:, W + 1:W + 2, :] = xp_ref[:, W - 1:W, :]

        for dy in range(3):
            for dx in range(3):
                t = dy * 3 + dx
                col_ref[:, t * C:(t + 1) * C] = (
                    xp_ref[dy:dy + th, dx:dx + W, :].reshape(th * W, C))

        y = jnp.dot(col_ref[...], w_ref[...],
                    preferred_element_type=jnp.float32)
        outl_ref[0] = y[:, :ocl]
        l2g_ref[0] = y[:, ocl:]

    return body


def _conv3x3_dual(x_l, x_g, wc, ocl, ocg, th=16):
    B, H, W, cl = x_l.shape
    cg = x_g.shape[-1]
    C = cl + cg
    n_th = H // th
    thb = th // 8

    outl, l2g = pl.pallas_call(
        _make_conv_body(th, W, cl, cg, ocl),
        out_shape=(jax.ShapeDtypeStruct((B, H * W, ocl), jnp.float32),
                   jax.ShapeDtypeStruct((B, H * W, ocg), jnp.float32)),
        grid_spec=pltpu.PrefetchScalarGridSpec(
            num_scalar_prefetch=0,
            grid=(B, n_th),
            in_specs=[
                pl.BlockSpec((1, th, W, cl), lambda b, i: (b, i, 0, 0)),
                pl.BlockSpec((1, th, W, cg), lambda b, i: (b, i, 0, 0)),
                pl.BlockSpec((1, 8, W, cl),
                             lambda b, i: (b, jnp.maximum(i * thb - 1, 0), 0, 0)),
                pl.BlockSpec((1, 8, W, cg),
                             lambda b, i: (b, jnp.maximum(i * thb - 1, 0), 0, 0)),
                pl.BlockSpec((1, 8, W, cl),
                             lambda b, i: (b, jnp.minimum((i + 1) * thb,
                                                          H // 8 - 1), 0, 0)),
                pl.BlockSpec((1, 8, W, cg),
                             lambda b, i: (b, jnp.minimum((i + 1) * thb,
                                                          H // 8 - 1), 0, 0)),
                pl.BlockSpec((9 * C, ocl + ocg), lambda b, i: (0, 0)),
            ],
            out_specs=[
                pl.BlockSpec((1, th * W, ocl), lambda b, i: (b, i, 0)),
                pl.BlockSpec((1, th * W, ocg), lambda b, i: (b, i, 0)),
            ],
            scratch_shapes=[
                pltpu.VMEM((th + 2, W + 2, C), jnp.bfloat16),
                pltpu.VMEM((th * W, 9 * C), jnp.bfloat16),
            ],
        ),
        compiler_params=pltpu.CompilerParams(
            dimension_semantics=("parallel", "parallel"),
            vmem_limit_bytes=96 << 20),
    )(x_l, x_g, x_l, x_g, x_l, x_g, wc)
    return (outl.reshape(B, H, W, ocl), l2g.reshape(B, H, W, ocg))


def _pw_body(x_ref, w_ref, sb_ref, out_ref):
    y = jnp.dot(x_ref[...].astype(jnp.bfloat16), w_ref[...],
                preferred_element_type=jnp.float32)
    y = y * sb_ref[0:1] + sb_ref[1:2]
    out_ref[...] = jnp.maximum(y, 0.0)


def _pw_affine_relu(x, w, scale, bias, tm=1024):
    lead = x.shape[:-1]
    Cin = x.shape[-1]
    Cout = w.shape[-1]
    M = int(math.prod(lead))
    grid = -(-M // tm)
    sb = jnp.stack([scale, bias]).astype(jnp.float32)
    out = pl.pallas_call(
        _pw_body,
        out_shape=jax.ShapeDtypeStruct((M, Cout), jnp.float32),
        grid_spec=pltpu.PrefetchScalarGridSpec(
            num_scalar_prefetch=0,
            grid=(grid,),
            in_specs=[
                pl.BlockSpec((tm, Cin), lambda i: (i, 0)),
                pl.BlockSpec((Cin, Cout), lambda i: (0, 0)),
                pl.BlockSpec((2, Cout), lambda i: (0, 0)),
            ],
            out_specs=pl.BlockSpec((tm, Cout), lambda i: (i, 0)),
        ),
        compiler_params=pltpu.CompilerParams(
            dimension_semantics=("parallel",)),
    )(x.reshape(M, Cin), w.astype(jnp.bfloat16), sb)
    return out.reshape(lead + (Cout,))


def _conv2_body(y_ref, fu_ref, xs_ref, l2g_ref, w_ref, out_ref):
    s = y_ref[0] + fu_ref[0] + xs_ref[0]
    Hh, Wh, c = s.shape
    o = jnp.dot(s.reshape(Hh * Wh, c).astype(jnp.bfloat16), w_ref[...],
                preferred_element_type=jnp.float32)
    o = o + l2g_ref[0].reshape(Hh * Wh, o.shape[-1])
    out_ref[0] = o.reshape(Hh, Wh, o.shape[-1])


def _conv2_fused(y, fu, xs_small, l2g, w):
    B, H, W, c = y.shape
    Cout = w.shape[-1]
    Hh, Wh = H // 2, W // 2
    out = pl.pallas_call(
        _conv2_body,
        out_shape=jax.ShapeDtypeStruct((B, H, W, Cout), jnp.float32),
        grid_spec=pltpu.PrefetchScalarGridSpec(
            num_scalar_prefetch=0,
            grid=(B, 2, 2),
            in_specs=[
                pl.BlockSpec((1, Hh, Wh, c), lambda b, i, j: (b, i, j, 0)),
                pl.BlockSpec((1, Hh, Wh, c), lambda b, i, j: (b, i, j, 0)),
                pl.BlockSpec((1, Hh, Wh, c), lambda b, i, j: (b, 0, 0, 0)),
                pl.BlockSpec((1, Hh, Wh, Cout), lambda b, i, j: (b, i, j, 0)),
                pl.BlockSpec((c, Cout), lambda b, i, j: (0, 0)),
            ],
            out_specs=pl.BlockSpec((1, Hh, Wh, Cout),
                                   lambda b, i, j: (b, i, j, 0)),
        ),
        compiler_params=pltpu.CompilerParams(
            dimension_semantics=("parallel", "parallel", "parallel")),
    )(y, fu, xs_small, l2g, w.astype(jnp.bfloat16))
    return out


def _dft_mats(Hh, Ww, Vp):
    import numpy as np
    V = Ww // 2 + 1
    u = np.arange(Hh)
    th = 2.0 * np.pi * np.outer(u, u) / Hh
    fhr = np.cos(th) / np.sqrt(Hh)
    fhi = -np.sin(th) / np.sqrt(Hh)
    w = np.arange(Ww)
    v = np.arange(Vp)
    ph = 2.0 * np.pi * np.outer(w, v) / Ww
    mask = (v < V).astype(np.float64)
    fwr = np.cos(ph) / np.sqrt(Ww) * mask
    fwi = -np.sin(ph) / np.sqrt(Ww) * mask
    ihr = np.cos(th) / np.sqrt(Hh)
    ihi = np.sin(th) / np.sqrt(Hh)
    alpha = np.where((v == 0) | (v == Ww // 2), 1.0, 2.0) * mask
    pw = 2.0 * np.pi * np.outer(v, w) / Ww
    icw = alpha[:, None] * np.cos(pw) / np.sqrt(Ww)
    isw = -alpha[:, None] * np.sin(pw) / np.sqrt(Ww)
    return [jnp.asarray(m, jnp.bfloat16)
            for m in (fhr, fhi, fwr, fwi, ihr, ihi, icw, isw)]


def _bspec(shape, blocked_dims=1):
    if blocked_dims == 0:
        return pl.BlockSpec(shape, lambda b: tuple(0 for _ in shape))
    return pl.BlockSpec((1,) + shape[1:],
                        lambda b: (b,) + tuple(0 for _ in shape[1:]))


def _batch_call(body, ins, blocked, out_shapes):
    B = ins[0].shape[0]
    return pl.pallas_call(
        body,
        out_shape=tuple(jax.ShapeDtypeStruct(s, jnp.float32)
                        for s in out_shapes),
        grid_spec=pltpu.PrefetchScalarGridSpec(
            num_scalar_prefetch=0,
            grid=(B,),
            in_specs=[_bspec(a.shape, d) for a, d in zip(ins, blocked)],
            out_specs=[_bspec(s) for s in out_shapes],
        ),
        compiler_params=pltpu.CompilerParams(
            dimension_semantics=("parallel",)),
    )(*ins)


_BF = jnp.bfloat16
_F32 = jnp.float32


def _fwdw_body(yt_ref, tt_ref, fwr, fwi, gwr, gwi,
               cr_ref, ci_ref, lr_ref, li_ref):
    yt = yt_ref[0]
    cr_ref[0] = jnp.dot(yt, fwr[...], preferred_element_type=_F32)
    ci_ref[0] = jnp.dot(yt, fwi[...], preferred_element_type=_F32)
    tt = tt_ref[0]
    lr_ref[0] = jnp.dot(tt, gwr[...], preferred_element_type=_F32)
    li_ref[0] = jnp.dot(tt, gwi[...], preferred_element_type=_F32)


def _fwdh_body(cr_ref, ci_ref, lr_ref, li_ref, fhr, fhi, ghr, ghi,
               fr_ref, fi_ref, gr_ref, gi_ref):
    cr = cr_ref[0].astype(_BF)
    ci = ci_ref[0].astype(_BF)
    fr_ref[0] = (jnp.dot(fhr[...], cr, preferred_element_type=_F32)
                 - jnp.dot(fhi[...], ci, preferred_element_type=_F32))
    fi_ref[0] = (jnp.dot(fhr[...], ci, preferred_element_type=_F32)
                 + jnp.dot(fhi[...], cr, preferred_element_type=_F32))
    lr = lr_ref[0].astype(_BF)
    li = li_ref[0].astype(_BF)
    gr_ref[0] = (jnp.dot(ghr[...], lr, preferred_element_type=_F32)
                 - jnp.dot(ghi[...], li, preferred_element_type=_F32))
    gi_ref[0] = (jnp.dot(ghr[...], li, preferred_element_type=_F32)
                 + jnp.dot(ghi[...], lr, preferred_element_type=_F32))


def _make_freqconv_body(cc):
    def body(fr_ref, fi_ref, lr_ref, li_ref, wfu, wlfu, sbf, sbl,
             gr_ref, gi_ref, hr_ref, hi_ref):
        fr = fr_ref[0].astype(_BF)
        fi = fi_ref[0].astype(_BF)
        g = (jnp.dot(fr, wfu[:cc], preferred_element_type=_F32)
             + jnp.dot(fi, wfu[cc:], preferred_element_type=_F32))
        g = jnp.maximum(g * sbf[0:1] + sbf[1:2], 0.0)
        gr_ref[0] = g[:, :cc]
        gi_ref[0] = g[:, cc:]
        lr = lr_ref[0].astype(_BF)
        li = li_ref[0].astype(_BF)
        h = (jnp.dot(lr, wlfu[:cc], preferred_element_type=_F32)
             + jnp.dot(li, wlfu[cc:], preferred_element_type=_F32))
        h = jnp.maximum(h * sbl[0:1] + sbl[1:2], 0.0)
        hr_ref[0] = h[:, :cc]
        hi_ref[0] = h[:, cc:]
    return body


def _invw_body(dr_ref, di_ref, ldr_ref, ldi_ref, icw, isw, jcw, jsw,
               fu_ref, xs_ref):
    dr = dr_ref[0].astype(_BF)
    di = di_ref[0].astype(_BF)
    fu_ref[0] = (jnp.dot(dr, icw[...], preferred_element_type=_F32)
                 + jnp.dot(di, isw[...], preferred_element_type=_F32))
    ldr = ldr_ref[0].astype(_BF)
    ldi = ldi_ref[0].astype(_BF)
    xs_ref[0] = (jnp.dot(ldr, jcw[...], preferred_element_type=_F32)
                 + jnp.dot(ldi, jsw[...], preferred_element_type=_F32))


def _spectral_pipeline(x_g, l2g, w1, w_fu, w_lfu, w2, s1, b1, sbfu, sblfu):
    B, H, W, cg = x_g.shape
    c = w1.shape[-1]
    H2, W2 = H // 2, W // 2
    c4 = c // 4
    Vp = ((W // 2 + 1) + 7) // 8 * 8
    Vp2 = ((W2 // 2 + 1) + 7) // 8 * 8
    fhr, fhi, fwr, fwi, ihr, ihi, icw, isw = _dft_mats(H, W, Vp)
    ghr, ghi, gwr, gwi, jhr, jhi, jcw, jsw = _dft_mats(H2, W2, Vp2)

    y = _pw_affine_relu(x_g, w1, s1, b1)

    yt = jnp.transpose(y, (0, 3, 1, 2)).astype(_BF)
    tt = jnp.concatenate(
        [yt[:, :c4, :H2, :W2], yt[:, :c4, H2:, :W2],
         yt[:, :c4, :H2, W2:], yt[:, :c4, H2:, W2:]], axis=1)

    cr, ci, lr, li = _batch_call(
        _fwdw_body,
        [yt.reshape(B, c * H, W), tt.reshape(B, c * H2, W2),
         fwr, fwi, gwr, gwi],
        [1, 1, 0, 0, 0, 0],
        [(B, c * H, Vp), (B, c * H, Vp), (B, c * H2, Vp2), (B, c * H2, Vp2)])

    tr = lambda a, cc, hh, vp: jnp.transpose(
        a.reshape(B, cc, hh, vp), (0, 2, 1, 3)).reshape(B, hh, cc * vp)
    fr, fi, glr, gli = _batch_call(
        _fwdh_body,
        [tr(cr, c, H, Vp), tr(ci, c, H, Vp),
         tr(lr, c, H2, Vp2), tr(li, c, H2, Vp2),
         fhr, fhi, ghr, ghi],
        [1, 1, 1, 1, 0, 0, 0, 0],
        [(B, H, c * Vp), (B, H, c * Vp),
         (B, H2, c * Vp2), (B, H2, c * Vp2)])

    tv = lambda a, hh, vp: jnp.transpose(
        a.reshape(B, hh, c, vp), (0, 1, 3, 2)).reshape(B, hh * vp, c)
    gr, gi, hr, hi = _batch_call(
        _make_freqconv_body(c),
        [tv(fr, H, Vp), tv(fi, H, Vp), tv(glr, H2, Vp2), tv(gli, H2, Vp2),
         w_fu.astype(_BF), w_lfu.astype(_BF), sbfu, sblfu],
        [1, 1, 1, 1, 0, 0, 0, 0],
        [(B, H * Vp, c), (B, H * Vp, c),
         (B, H2 * Vp2, c), (B, H2 * Vp2, c)])

    rs = lambda a, hh, vp: a.reshape(B, hh, vp * c)
    dr, di, ldr, ldi = _batch_call(
        _fwdh_body,
        [rs(gr, H, Vp), rs(gi, H, Vp), rs(hr, H2, Vp2), rs(hi, H2, Vp2),
         ihr, ihi, jhr, jhi],
        [1, 1, 1, 1, 0, 0, 0, 0],
        [(B, H, Vp * c), (B, H, Vp * c),
         (B, H2, Vp2 * c), (B, H2, Vp2 * c)])

    tc = lambda a, hh, vp: jnp.transpose(
        a.reshape(B, hh, vp, c), (0, 1, 3, 2)).reshape(B, hh * c, vp)
    fu_t, xs_t = _batch_call(
        _invw_body,
        [tc(dr, H, Vp), tc(di, H, Vp), tc(ldr, H2, Vp2), tc(ldi, H2, Vp2),
         icw, isw, jcw, jsw],
        [1, 1, 1, 1, 0, 0, 0, 0],
        [(B, H * c, W), (B, H2 * c, W2)])

    fu = jnp.transpose(fu_t.reshape(B, H, c, W), (0, 1, 3, 2))
    xs = jnp.transpose(xs_t.reshape(B, H2, c, W2), (0, 1, 3, 2))
    return _conv2_fused(y, fu, xs, l2g, w2)


def _lfu_fold(y):
    B, H, W, c = y.shape
    c4 = c // 4
    t = y[..., :c4]
    t = jnp.concatenate([t[:, : H // 2], t[:, H // 2:]], axis=-1)
    t = jnp.concatenate([t[:, :, : W // 2], t[:, :, W // 2:]], axis=-1)
    return t


def _fourier_unit(t, w, gamma, beta, mean, var):
    Hh, Ww, cch = t.shape[1], t.shape[2], t.shape[3]
    f = jnp.fft.rfft2(t, axes=(1, 2), norm="ortho")
    fr = jnp.concatenate([f.real, f.imag], axis=-1).astype(jnp.float32)
    s, b = _bn_scale_bias(gamma, beta, mean, var)
    g = _pw_affine_relu(fr, w, s, b)
    gc = lax.complex(g[..., :cch], g[..., cch:])
    return jnp.fft.irfft2(gc, s=(Hh, Ww), axes=(1, 2),
                          norm="ortho").astype(jnp.float32)


def kernel(x_l, x_g, w_l2l, w_g2l, w_l2g, w1, w_fu, w_lfu, w2,
           bn1_gamma, bn1_beta, bn1_mean, bn1_var,
           fu_bn_gamma, fu_bn_beta, fu_bn_mean, fu_bn_var,
           lfu_bn_gamma, lfu_bn_beta, lfu_bn_mean, lfu_bn_var):
    B, H, W, cl = x_l.shape
    cg = x_g.shape[-1]
    ocl = w_l2l.shape[-1]
    ocg = w_l2g.shape[-1]
    C = cl + cg

    wc = jnp.zeros((3, 3, C, ocl + ocg), jnp.float32)
    wc = wc.at[:, :, :cl, :ocl].set(w_l2l)
    wc = wc.at[:, :, cl:, :ocl].set(w_g2l)
    wc = wc.at[:, :, :cl, ocl:].set(w_l2g)
    wc = wc.reshape(9 * C, ocl + ocg).astype(jnp.bfloat16)

    out_l, l2g = _conv3x3_dual(x_l, x_g, wc, ocl, ocg)

    s1, b1 = _bn_scale_bias(bn1_gamma, bn1_beta, bn1_mean, bn1_var)
    sfu, bfu = _bn_scale_bias(fu_bn_gamma, fu_bn_beta, fu_bn_mean, fu_bn_var)
    slf, blf = _bn_scale_bias(lfu_bn_gamma, lfu_bn_beta, lfu_bn_mean, lfu_bn_var)
    sb1 = jnp.stack([s1, b1]).astype(jnp.float32)
    sbfu = jnp.stack([sfu, bfu]).astype(jnp.float32)
    sblfu = jnp.stack([slf, blf]).astype(jnp.float32)

    out_g = _spectral_pipeline(x_g, l2g, w1, w_fu, w_lfu, w2,
                               s1, b1, sbfu, sblfu)
    return out_l, out_g


# speedup vs baseline: 1.4947x; 1.4947x over previous
import math

import jax
import jax.numpy as jnp
from jax import lax
from jax.experimental import pallas as pl
from jax.experimental.pallas import tpu as pltpu


def _bn_scale_bias(gamma, beta, mean, var, eps=1e-5):
    s = gamma / jnp.sqrt(var + eps)
    return s, beta - mean * s


def _make_conv_body(th, W, cl, cg, ocl):
    C = cl + cg
    bf = jnp.bfloat16

    def body(xl_ref, xg_ref, tl_ref, tg_ref, bl_ref, bg_ref, w_ref,
             outl_ref, l2g_ref, xp_ref, col_ref):
        i = pl.program_id(1)
        n = pl.num_programs(1)

        xp_ref[1:th + 1, 1:W + 1, :cl] = xl_ref[0].astype(bf)
        xp_ref[1:th + 1, 1:W + 1, cl:] = xg_ref[0].astype(bf)

        @pl.when(i == 0)
        def _():
            xp_ref[0:1, 1:W + 1, :cl] = xl_ref[0, 1:2].astype(bf)
            xp_ref[0:1, 1:W + 1, cl:] = xg_ref[0, 1:2].astype(bf)

        @pl.when(i > 0)
        def _():
            xp_ref[0:1, 1:W + 1, :cl] = tl_ref[0, 7:8].astype(bf)
            xp_ref[0:1, 1:W + 1, cl:] = tg_ref[0, 7:8].astype(bf)

        @pl.when(i == n - 1)
        def _():
            xp_ref[th + 1:th + 2, 1:W + 1, :cl] = xl_ref[0, th - 2:th - 1].astype(bf)
            xp_ref[th + 1:th + 2, 1:W + 1, cl:] = xg_ref[0, th - 2:th - 1].astype(bf)

        @pl.when(i < n - 1)
        def _():
            xp_ref[th + 1:th + 2, 1:W + 1, :cl] = bl_ref[0, 0:1].astype(bf)
            xp_ref[th + 1:th + 2, 1:W + 1, cl:] = bg_ref[0, 0:1].astype(bf)

        xp_ref[:, 0:1, :] = xp_ref[:, 2:3, :]
        xp_ref[:, W + 1:W + 2, :] = xp_ref[:, W - 1:W, :]

        for dy in range(3):
            for dx in range(3):
                t = dy * 3 + dx
                col_ref[:, t * C:(t + 1) * C] = (
                    xp_ref[dy:dy + th, dx:dx + W, :].reshape(th * W, C))

        y = jnp.dot(col_ref[...], w_ref[...],
                    preferred_element_type=jnp.float32)
        outl_ref[0] = y[:, :ocl]
        l2g_ref[0] = y[:, ocl:]

    return body


def _conv3x3_dual(x_l, x_g, wc, ocl, ocg, th=16):
    B, H, W, cl = x_l.shape
    cg = x_g.shape[-1]
    C = cl + cg
    n_th = H // th
    thb = th // 8

    outl, l2g = pl.pallas_call(
        _make_conv_body(th, W, cl, cg, ocl),
        out_shape=(jax.ShapeDtypeStruct((B, H * W, ocl), jnp.float32),
                   jax.ShapeDtypeStruct((B, H * W, ocg), jnp.float32)),
        grid_spec=pltpu.PrefetchScalarGridSpec(
            num_scalar_prefetch=0,
            grid=(B, n_th),
            in_specs=[
                pl.BlockSpec((1, th, W, cl), lambda b, i: (b, i, 0, 0)),
                pl.BlockSpec((1, th, W, cg), lambda b, i: (b, i, 0, 0)),
                pl.BlockSpec((1, 8, W, cl),
                             lambda b, i: (b, jnp.maximum(i * thb - 1, 0), 0, 0)),
                pl.BlockSpec((1, 8, W, cg),
                             lambda b, i: (b, jnp.maximum(i * thb - 1, 0), 0, 0)),
                pl.BlockSpec((1, 8, W, cl),
                             lambda b, i: (b, jnp.minimum((i + 1) * thb,
                                                          H // 8 - 1), 0, 0)),
                pl.BlockSpec((1, 8, W, cg),
                             lambda b, i: (b, jnp.minimum((i + 1) * thb,
                                                          H // 8 - 1), 0, 0)),
                pl.BlockSpec((9 * C, ocl + ocg), lambda b, i: (0, 0)),
            ],
            out_specs=[
                pl.BlockSpec((1, th * W, ocl), lambda b, i: (b, i, 0)),
                pl.BlockSpec((1, th * W, ocg), lambda b, i: (b, i, 0)),
            ],
            scratch_shapes=[
                pltpu.VMEM((th + 2, W + 2, C), jnp.bfloat16),
                pltpu.VMEM((th * W, 9 * C), jnp.bfloat16),
            ],
        ),
        compiler_params=pltpu.CompilerParams(
            dimension_semantics=("parallel", "parallel"),
            vmem_limit_bytes=96 << 20),
    )(x_l, x_g, x_l, x_g, x_l, x_g, wc)
    return (outl.reshape(B, H, W, ocl), l2g.reshape(B, H, W, ocg))


def _pw_body(x_ref, w_ref, sb_ref, out_ref):
    y = jnp.dot(x_ref[...].astype(jnp.bfloat16), w_ref[...],
                preferred_element_type=jnp.float32)
    y = y * sb_ref[0:1] + sb_ref[1:2]
    out_ref[...] = jnp.maximum(y, 0.0)


def _pw_affine_relu(x, w, scale, bias, tm=1024):
    lead = x.shape[:-1]
    Cin = x.shape[-1]
    Cout = w.shape[-1]
    M = int(math.prod(lead))
    grid = -(-M // tm)
    sb = jnp.stack([scale, bias]).astype(jnp.float32)
    out = pl.pallas_call(
        _pw_body,
        out_shape=jax.ShapeDtypeStruct((M, Cout), jnp.float32),
        grid_spec=pltpu.PrefetchScalarGridSpec(
            num_scalar_prefetch=0,
            grid=(grid,),
            in_specs=[
                pl.BlockSpec((tm, Cin), lambda i: (i, 0)),
                pl.BlockSpec((Cin, Cout), lambda i: (0, 0)),
                pl.BlockSpec((2, Cout), lambda i: (0, 0)),
            ],
            out_specs=pl.BlockSpec((tm, Cout), lambda i: (i, 0)),
        ),
        compiler_params=pltpu.CompilerParams(
            dimension_semantics=("parallel",)),
    )(x.reshape(M, Cin), w.astype(jnp.bfloat16), sb)
    return out.reshape(lead + (Cout,))


def _conv2_body(y_ref, fu_ref, xs_ref, l2g_ref, w_ref, out_ref):
    s = (y_ref[0].astype(jnp.float32) + fu_ref[0].astype(jnp.float32)
         + xs_ref[0].astype(jnp.float32))
    Hh, Wh, c = s.shape
    o = jnp.dot(s.reshape(Hh * Wh, c).astype(jnp.bfloat16), w_ref[...],
                preferred_element_type=jnp.float32)
    o = o + l2g_ref[0].reshape(Hh * Wh, o.shape[-1])
    out_ref[0] = o.reshape(Hh, Wh, o.shape[-1])


def _conv2_fused(y, fu, xs_small, l2g, w):
    B, H, W, c = y.shape
    Cout = w.shape[-1]
    Hh, Wh = H // 2, W // 2
    out = pl.pallas_call(
        _conv2_body,
        out_shape=jax.ShapeDtypeStruct((B, H, W, Cout), jnp.float32),
        grid_spec=pltpu.PrefetchScalarGridSpec(
            num_scalar_prefetch=0,
            grid=(B, 2, 2),
            in_specs=[
                pl.BlockSpec((1, Hh, Wh, c), lambda b, i, j: (b, i, j, 0)),
                pl.BlockSpec((1, Hh, Wh, c), lambda b, i, j: (b, i, j, 0)),
                pl.BlockSpec((1, Hh, Wh, c), lambda b, i, j: (b, 0, 0, 0)),
                pl.BlockSpec((1, Hh, Wh, Cout), lambda b, i, j: (b, i, j, 0)),
                pl.BlockSpec((c, Cout), lambda b, i, j: (0, 0)),
            ],
            out_specs=pl.BlockSpec((1, Hh, Wh, Cout),
                                   lambda b, i, j: (b, i, j, 0)),
        ),
        compiler_params=pltpu.CompilerParams(
            dimension_semantics=("parallel", "parallel", "parallel")),
    )(y, fu, xs_small, l2g, w.astype(jnp.bfloat16))
    return out


def _dft_mats(Hh, Ww, Vp):
    import numpy as np
    V = Ww // 2 + 1
    u = np.arange(Hh)
    th = 2.0 * np.pi * np.outer(u, u) / Hh
    fhr = np.cos(th) / np.sqrt(Hh)
    fhi = -np.sin(th) / np.sqrt(Hh)
    w = np.arange(Ww)
    v = np.arange(Vp)
    ph = 2.0 * np.pi * np.outer(w, v) / Ww
    mask = (v < V).astype(np.float64)
    fwr = np.cos(ph) / np.sqrt(Ww) * mask
    fwi = -np.sin(ph) / np.sqrt(Ww) * mask
    ihr = np.cos(th) / np.sqrt(Hh)
    ihi = np.sin(th) / np.sqrt(Hh)
    alpha = np.where((v == 0) | (v == Ww // 2), 1.0, 2.0) * mask
    pw = 2.0 * np.pi * np.outer(v, w) / Ww
    icw = alpha[:, None] * np.cos(pw) / np.sqrt(Ww)
    isw = -alpha[:, None] * np.sin(pw) / np.sqrt(Ww)
    return [jnp.asarray(m, jnp.bfloat16)
            for m in (fhr, fhi, fwr, fwi, ihr, ihi, icw, isw)]


def _bspec(shape, blocked_dims=1):
    if blocked_dims == 0:
        return pl.BlockSpec(shape, lambda b: tuple(0 for _ in shape))
    return pl.BlockSpec((1,) + shape[1:],
                        lambda b: (b,) + tuple(0 for _ in shape[1:]))


def _batch_call(body, ins, blocked, out_shapes, out_dtype=jnp.bfloat16):
    B = ins[0].shape[0]
    return pl.pallas_call(
        body,
        out_shape=tuple(jax.ShapeDtypeStruct(s, out_dtype)
                        for s in out_shapes),
        grid_spec=pltpu.PrefetchScalarGridSpec(
            num_scalar_prefetch=0,
            grid=(B,),
            in_specs=[_bspec(a.shape, d) for a, d in zip(ins, blocked)],
            out_specs=[_bspec(s) for s in out_shapes],
        ),
        compiler_params=pltpu.CompilerParams(
            dimension_semantics=("parallel",)),
    )(*ins)


_BF = jnp.bfloat16
_F32 = jnp.float32


def _mxt(x_bf, i_ref):
    return lax.dot_general(i_ref[...], x_bf, (((1,), (1,)), ((), ())),
                           preferred_element_type=_F32)


def _make_pw2_body(HW, cg):
    def body(x_ref, w_ref, sb_ref, ic_ref, y_ref, yt_ref):
        x = x_ref[0].reshape(HW, cg).astype(_BF)
        y = jnp.dot(x, w_ref[...], preferred_element_type=_F32)
        y = jnp.maximum(y * sb_ref[0:1] + sb_ref[1:2], 0.0)
        yb = y.astype(_BF)
        y_ref[0] = yb
        yt_ref[0] = _mxt(yb, ic_ref).astype(_BF)
    return body


def _fwdw_body(yt_ref, tt_ref, fwr, fwi, gwr, gwi, iv, iv2,
               crt_ref, cit_ref, lrt_ref, lit_ref):
    yt = yt_ref[0]
    zr = jnp.dot(yt, fwr[...], preferred_element_type=_F32)
    zi = jnp.dot(yt, fwi[...], preferred_element_type=_F32)
    crt_ref[0] = _mxt(zr.astype(_BF), iv).astype(_BF)
    cit_ref[0] = _mxt(zi.astype(_BF), iv).astype(_BF)
    tt = tt_ref[0]
    wr = jnp.dot(tt, gwr[...], preferred_element_type=_F32)
    wi = jnp.dot(tt, gwi[...], preferred_element_type=_F32)
    lrt_ref[0] = _mxt(wr.astype(_BF), iv2).astype(_BF)
    lit_ref[0] = _mxt(wi.astype(_BF), iv2).astype(_BF)


def _fwdh_body(cr_ref, ci_ref, lr_ref, li_ref, fhr, fhi, ghr, ghi, iu, iu2,
               fr_ref, fi_ref, gr_ref, gi_ref):
    cr = cr_ref[0]
    ci = ci_ref[0]
    fr = (jnp.dot(cr, fhr[...], preferred_element_type=_F32)
          - jnp.dot(ci, fhi[...], preferred_element_type=_F32))
    fi = (jnp.dot(cr, fhi[...], preferred_element_type=_F32)
          + jnp.dot(ci, fhr[...], preferred_element_type=_F32))
    fr_ref[0] = _mxt(fr.astype(_BF), iu).astype(_BF)
    fi_ref[0] = _mxt(fi.astype(_BF), iu).astype(_BF)
    lr = lr_ref[0]
    li = li_ref[0]
    gr = (jnp.dot(lr, ghr[...], preferred_element_type=_F32)
          - jnp.dot(li, ghi[...], preferred_element_type=_F32))
    gi = (jnp.dot(lr, ghi[...], preferred_element_type=_F32)
          + jnp.dot(li, ghr[...], preferred_element_type=_F32))
    gr_ref[0] = _mxt(gr.astype(_BF), iu2).astype(_BF)
    gi_ref[0] = _mxt(gi.astype(_BF), iu2).astype(_BF)


def _make_freqconv_body(cc):
    def body(fr_ref, fi_ref, lr_ref, li_ref, wfu, wlfu, sbf, sbl,
             gr_ref, gi_ref, hr_ref, hi_ref):
        fr = fr_ref[0]
        fi = fi_ref[0]
        g = (jnp.dot(fr, wfu[:cc], preferred_element_type=_F32)
             + jnp.dot(fi, wfu[cc:], preferred_element_type=_F32))
        g = jnp.maximum(g * sbf[0:1] + sbf[1:2], 0.0)
        gr_ref[0] = g[:, :cc].astype(_BF)
        gi_ref[0] = g[:, cc:].astype(_BF)
        lr = lr_ref[0]
        li = li_ref[0]
        h = (jnp.dot(lr, wlfu[:cc], preferred_element_type=_F32)
             + jnp.dot(li, wlfu[cc:], preferred_element_type=_F32))
        h = jnp.maximum(h * sbl[0:1] + sbl[1:2], 0.0)
        hr_ref[0] = h[:, :cc].astype(_BF)
        hi_ref[0] = h[:, cc:].astype(_BF)
    return body


def _invh_body(gr_ref, gi_ref, hr_ref, hi_ref, ihr, ihi, jhr, jhi,
               dr_ref, di_ref, er_ref, ei_ref):
    gr = gr_ref[0]
    gi = gi_ref[0]
    dr_ref[0] = (jnp.dot(ihr[...], gr, preferred_element_type=_F32)
                 - jnp.dot(ihi[...], gi, preferred_element_type=_F32)
                 ).astype(_BF)
    di_ref[0] = (jnp.dot(ihr[...], gi, preferred_element_type=_F32)
                 + jnp.dot(ihi[...], gr, preferred_element_type=_F32)
                 ).astype(_BF)
    hr = hr_ref[0]
    hi = hi_ref[0]
    er_ref[0] = (jnp.dot(jhr[...], hr, preferred_element_type=_F32)
                 - jnp.dot(jhi[...], hi, preferred_element_type=_F32)
                 ).astype(_BF)
    ei_ref[0] = (jnp.dot(jhr[...], hi, preferred_element_type=_F32)
                 + jnp.dot(jhi[...], hr, preferred_element_type=_F32)
                 ).astype(_BF)


def _invw_body(dr_ref, di_ref, ldr_ref, ldi_ref, icw, isw, jcw, jsw,
               fu_ref, xs_ref):
    dr = dr_ref[0]
    di = di_ref[0]
    fu_ref[0] = (jnp.dot(dr, icw[...], preferred_element_type=_F32)
                 + jnp.dot(di, isw[...], preferred_element_type=_F32)
                 ).astype(_BF)
    ldr = ldr_ref[0]
    ldi = ldi_ref[0]
    xs_ref[0] = (jnp.dot(ldr, jcw[...], preferred_element_type=_F32)
                 + jnp.dot(ldi, jsw[...], preferred_element_type=_F32)
                 ).astype(_BF)


def _spectral_pipeline(x_g, l2g, w1, w_fu, w_lfu, w2, sb1, sbfu, sblfu):
    B, H, W, cg = x_g.shape
    c = w1.shape[-1]
    H2, W2 = H // 2, W // 2
    c4 = c // 4
    Vp = ((W // 2 + 1) + 7) // 8 * 8
    Vp2 = ((W2 // 2 + 1) + 7) // 8 * 8
    fhr, fhi, fwr, fwi, ihr, ihi, icw, isw = _dft_mats(H, W, Vp)
    ghr, ghi, gwr, gwi, jhr, jhi, jcw, jsw = _dft_mats(H2, W2, Vp2)
    ic = jnp.eye(c, dtype=_BF)
    iv, iv2 = jnp.eye(Vp, dtype=_BF), jnp.eye(Vp2, dtype=_BF)
    iu, iu2 = jnp.eye(H, dtype=_BF), jnp.eye(H2, dtype=_BF)

    y, yt = _batch_call(
        _make_pw2_body(H * W, cg),
        [x_g, w1.astype(_BF), sb1, ic],
        [1, 0, 0, 0],
        [(B, H * W, c), (B, c, H * W)])

    yt4 = yt.reshape(B, c, H, W)
    tt = jnp.concatenate(
        [yt4[:, :c4, :H2, :W2], yt4[:, :c4, H2:, :W2],
         yt4[:, :c4, :H2, W2:], yt4[:, :c4, H2:, W2:]], axis=1)

    crt, cit, lrt, lit = _batch_call(
        _fwdw_body,
        [yt4.reshape(B, c * H, W), tt.reshape(B, c * H2, W2),
         fwr, fwi, gwr, gwi, iv, iv2],
        [1, 1, 0, 0, 0, 0, 0, 0],
        [(B, Vp, c * H), (B, Vp, c * H),
         (B, Vp2, c * H2), (B, Vp2, c * H2)])

    rs1 = lambda a, vp, hh: a.reshape(B, vp * c, hh)
    fr, fi, glr, gli = _batch_call(
        _fwdh_body,
        [rs1(crt, Vp, H), rs1(cit, Vp, H), rs1(lrt, Vp2, H2),
         rs1(lit, Vp2, H2), fhr, fhi, ghr, ghi, iu, iu2],
        [1, 1, 1, 1, 0, 0, 0, 0, 0, 0],
        [(B, H, Vp * c), (B, H, Vp * c),
         (B, H2, Vp2 * c), (B, H2, Vp2 * c)])

    rs2 = lambda a, hh, vp: a.reshape(B, hh * vp, c)
    gr, gi, hr, hi = _batch_call(
        _make_freqconv_body(c),
        [rs2(fr, H, Vp), rs2(fi, H, Vp), rs2(glr, H2, Vp2),
         rs2(gli, H2, Vp2), w_fu.astype(_BF), w_lfu.astype(_BF),
         sbfu, sblfu],
        [1, 1, 1, 1, 0, 0, 0, 0],
        [(B, H * Vp, c), (B, H * Vp, c),
         (B, H2 * Vp2, c), (B, H2 * Vp2, c)])

    rs3 = lambda a, hh, vp: a.reshape(B, hh, vp * c)
    dr, di, ldr, ldi = _batch_call(
        _invh_body,
        [rs3(gr, H, Vp), rs3(gi, H, Vp), rs3(hr, H2, Vp2),
         rs3(hi, H2, Vp2), ihr, ihi, jhr, jhi],
        [1, 1, 1, 1, 0, 0, 0, 0],
        [(B, H, Vp * c), (B, H, Vp * c),
         (B, H2, Vp2 * c), (B, H2, Vp2 * c)])

    tc = lambda a, hh, vp: jnp.transpose(
        a.reshape(B, hh, vp, c), (0, 1, 3, 2)).reshape(B, hh * c, vp)
    fu_t, xs_t = _batch_call(
        _invw_body,
        [tc(dr, H, Vp), tc(di, H, Vp), tc(ldr, H2, Vp2), tc(ldi, H2, Vp2),
         icw, isw, jcw, jsw],
        [1, 1, 1, 1, 0, 0, 0, 0],
        [(B, H * c, W), (B, H2 * c, W2)])

    fu = jnp.transpose(fu_t.reshape(B, H, c, W), (0, 1, 3, 2))
    xs = jnp.transpose(xs_t.reshape(B, H2, c, W2), (0, 1, 3, 2))
    return y, fu, xs


def _lfu_fold(y):
    B, H, W, c = y.shape
    c4 = c // 4
    t = y[..., :c4]
    t = jnp.concatenate([t[:, : H // 2], t[:, H // 2:]], axis=-1)
    t = jnp.concatenate([t[:, :, : W // 2], t[:, :, W // 2:]], axis=-1)
    return t


def _fourier_unit(t, w, gamma, beta, mean, var):
    Hh, Ww, cch = t.shape[1], t.shape[2], t.shape[3]
    f = jnp.fft.rfft2(t, axes=(1, 2), norm="ortho")
    fr = jnp.concatenate([f.real, f.imag], axis=-1).astype(jnp.float32)
    s, b = _bn_scale_bias(gamma, beta, mean, var)
    g = _pw_affine_relu(fr, w, s, b)
    gc = lax.complex(g[..., :cch], g[..., cch:])
    return jnp.fft.irfft2(gc, s=(Hh, Ww), axes=(1, 2),
                          norm="ortho").astype(jnp.float32)


def kernel(x_l, x_g, w_l2l, w_g2l, w_l2g, w1, w_fu, w_lfu, w2,
           bn1_gamma, bn1_beta, bn1_mean, bn1_var,
           fu_bn_gamma, fu_bn_beta, fu_bn_mean, fu_bn_var,
           lfu_bn_gamma, lfu_bn_beta, lfu_bn_mean, lfu_bn_var):
    B, H, W, cl = x_l.shape
    cg = x_g.shape[-1]
    ocl = w_l2l.shape[-1]
    ocg = w_l2g.shape[-1]
    C = cl + cg

    wc = jnp.zeros((3, 3, C, ocl + ocg), jnp.float32)
    wc = wc.at[:, :, :cl, :ocl].set(w_l2l)
    wc = wc.at[:, :, cl:, :ocl].set(w_g2l)
    wc = wc.at[:, :, :cl, ocl:].set(w_l2g)
    wc = wc.reshape(9 * C, ocl + ocg).astype(jnp.bfloat16)

    out_l, l2g = _conv3x3_dual(x_l, x_g, wc, ocl, ocg)

    s1, b1 = _bn_scale_bias(bn1_gamma, bn1_beta, bn1_mean, bn1_var)
    sfu, bfu = _bn_scale_bias(fu_bn_gamma, fu_bn_beta, fu_bn_mean, fu_bn_var)
    slf, blf = _bn_scale_bias(lfu_bn_gamma, lfu_bn_beta, lfu_bn_mean, lfu_bn_var)
    sb1 = jnp.stack([s1, b1]).astype(jnp.float32)
    sbfu = jnp.stack([sfu, bfu]).astype(jnp.float32)
    sblfu = jnp.stack([slf, blf]).astype(jnp.float32)

    y, fu, xs = _spectral_pipeline(x_g, l2g, w1, w_fu, w_lfu, w2,
                                   sb1, sbfu, sblfu)
    out_g = _conv2_fused(y.reshape(x_g.shape[:3] + (w1.shape[-1],)),
                         fu, xs, l2g, w2)
    return out_l, out_g


# speedup vs baseline: 1.5391x; 1.0297x over previous
import math

import jax
import jax.numpy as jnp
from jax import lax
from jax.experimental import pallas as pl
from jax.experimental.pallas import tpu as pltpu


def _bn_scale_bias(gamma, beta, mean, var, eps=1e-5):
    s = gamma / jnp.sqrt(var + eps)
    return s, beta - mean * s


def _make_conv_body(th, W, cl, cg, ocl):
    C = cl + cg
    bf = jnp.bfloat16

    def body(xl_ref, xg_ref, tl_ref, tg_ref, bl_ref, bg_ref, w_ref,
             outl_ref, l2g_ref, xp_ref, col_ref):
        i = pl.program_id(1)
        n = pl.num_programs(1)

        xp_ref[1:th + 1, 1:W + 1, :cl] = xl_ref[0].astype(bf)
        xp_ref[1:th + 1, 1:W + 1, cl:] = xg_ref[0].astype(bf)

        @pl.when(i == 0)
        def _():
            xp_ref[0:1, 1:W + 1, :cl] = xl_ref[0, 1:2].astype(bf)
            xp_ref[0:1, 1:W + 1, cl:] = xg_ref[0, 1:2].astype(bf)

        @pl.when(i > 0)
        def _():
            xp_ref[0:1, 1:W + 1, :cl] = tl_ref[0, 7:8].astype(bf)
            xp_ref[0:1, 1:W + 1, cl:] = tg_ref[0, 7:8].astype(bf)

        @pl.when(i == n - 1)
        def _():
            xp_ref[th + 1:th + 2, 1:W + 1, :cl] = xl_ref[0, th - 2:th - 1].astype(bf)
            xp_ref[th + 1:th + 2, 1:W + 1, cl:] = xg_ref[0, th - 2:th - 1].astype(bf)

        @pl.when(i < n - 1)
        def _():
            xp_ref[th + 1:th + 2, 1:W + 1, :cl] = bl_ref[0, 0:1].astype(bf)
            xp_ref[th + 1:th + 2, 1:W + 1, cl:] = bg_ref[0, 0:1].astype(bf)

        xp_ref[:, 0:1, :] = xp_ref[:, 2:3, :]
        xp_ref[:, W + 1:W + 2, :] = xp_ref[:, W - 1:W, :]

        for dy in range(3):
            for dx in range(3):
                t = dy * 3 + dx
                col_ref[:, t * C:(t + 1) * C] = (
                    xp_ref[dy:dy + th, dx:dx + W, :].reshape(th * W, C))

        y = jnp.dot(col_ref[...], w_ref[...],
                    preferred_element_type=jnp.float32)
        outl_ref[0] = y[:, :ocl]
        l2g_ref[0] = y[:, ocl:].astype(bf)

    return body


def _conv3x3_dual(x_l, x_g, wc, ocl, ocg, th=16):
    B, H, W, cl = x_l.shape
    cg = x_g.shape[-1]
    C = cl + cg
    n_th = H // th
    thb = th // 8

    outl, l2g = pl.pallas_call(
        _make_conv_body(th, W, cl, cg, ocl),
        out_shape=(jax.ShapeDtypeStruct((B, H * W, ocl), jnp.float32),
                   jax.ShapeDtypeStruct((B, H * W, ocg), jnp.bfloat16)),
        grid_spec=pltpu.PrefetchScalarGridSpec(
            num_scalar_prefetch=0,
            grid=(B, n_th),
            in_specs=[
                pl.BlockSpec((1, th, W, cl), lambda b, i: (b, i, 0, 0)),
                pl.BlockSpec((1, th, W, cg), lambda b, i: (b, i, 0, 0)),
                pl.BlockSpec((1, 8, W, cl),
                             lambda b, i: (b, jnp.maximum(i * thb - 1, 0), 0, 0)),
                pl.BlockSpec((1, 8, W, cg),
                             lambda b, i: (b, jnp.maximum(i * thb - 1, 0), 0, 0)),
                pl.BlockSpec((1, 8, W, cl),
                             lambda b, i: (b, jnp.minimum((i + 1) * thb,
                                                          H // 8 - 1), 0, 0)),
                pl.BlockSpec((1, 8, W, cg),
                             lambda b, i: (b, jnp.minimum((i + 1) * thb,
                                                          H // 8 - 1), 0, 0)),
                pl.BlockSpec((9 * C, ocl + ocg), lambda b, i: (0, 0)),
            ],
            out_specs=[
                pl.BlockSpec((1, th * W, ocl), lambda b, i: (b, i, 0)),
                pl.BlockSpec((1, th * W, ocg), lambda b, i: (b, i, 0)),
            ],
            scratch_shapes=[
                pltpu.VMEM((th + 2, W + 2, C), jnp.bfloat16),
                pltpu.VMEM((th * W, 9 * C), jnp.bfloat16),
            ],
        ),
        compiler_params=pltpu.CompilerParams(
            dimension_semantics=("parallel", "parallel"),
            vmem_limit_bytes=96 << 20),
    )(x_l, x_g, x_l, x_g, x_l, x_g, wc)
    return (outl.reshape(B, H, W, ocl), l2g.reshape(B, H, W, ocg))


def _pw_body(x_ref, w_ref, sb_ref, out_ref):
    y = jnp.dot(x_ref[...].astype(jnp.bfloat16), w_ref[...],
                preferred_element_type=jnp.float32)
    y = y * sb_ref[0:1] + sb_ref[1:2]
    out_ref[...] = jnp.maximum(y, 0.0)


def _pw_affine_relu(x, w, scale, bias, tm=1024):
    lead = x.shape[:-1]
    Cin = x.shape[-1]
    Cout = w.shape[-1]
    M = int(math.prod(lead))
    grid = -(-M // tm)
    sb = jnp.stack([scale, bias]).astype(jnp.float32)
    out = pl.pallas_call(
        _pw_body,
        out_shape=jax.ShapeDtypeStruct((M, Cout), jnp.float32),
        grid_spec=pltpu.PrefetchScalarGridSpec(
            num_scalar_prefetch=0,
            grid=(grid,),
            in_specs=[
                pl.BlockSpec((tm, Cin), lambda i: (i, 0)),
                pl.BlockSpec((Cin, Cout), lambda i: (0, 0)),
                pl.BlockSpec((2, Cout), lambda i: (0, 0)),
            ],
            out_specs=pl.BlockSpec((tm, Cout), lambda i: (i, 0)),
        ),
        compiler_params=pltpu.CompilerParams(
            dimension_semantics=("parallel",)),
    )(x.reshape(M, Cin), w.astype(jnp.bfloat16), sb)
    return out.reshape(lead + (Cout,))


def _conv2_body(y_ref, fu_ref, xs_ref, l2g_ref, w_ref, out_ref):
    s = (y_ref[0].astype(jnp.float32) + fu_ref[0].astype(jnp.float32)
         + xs_ref[0].astype(jnp.float32))
    Hh, Wh, c = s.shape
    o = jnp.dot(s.reshape(Hh * Wh, c).astype(jnp.bfloat16), w_ref[...],
                preferred_element_type=jnp.float32)
    o = o + l2g_ref[0].reshape(Hh * Wh, o.shape[-1])
    out_ref[0] = o.reshape(Hh, Wh, o.shape[-1])


def _conv2_fused(y, fu, xs_small, l2g, w):
    B, H, W, c = y.shape
    Cout = w.shape[-1]
    Hh, Wh = H // 2, W // 2
    out = pl.pallas_call(
        _conv2_body,
        out_shape=jax.ShapeDtypeStruct((B, H, W, Cout), jnp.float32),
        grid_spec=pltpu.PrefetchScalarGridSpec(
            num_scalar_prefetch=0,
            grid=(B, 2, 2),
            in_specs=[
                pl.BlockSpec((1, Hh, Wh, c), lambda b, i, j: (b, i, j, 0)),
                pl.BlockSpec((1, Hh, Wh, c), lambda b, i, j: (b, i, j, 0)),
                pl.BlockSpec((1, Hh, Wh, c), lambda b, i, j: (b, 0, 0, 0)),
                pl.BlockSpec((1, Hh, Wh, Cout), lambda b, i, j: (b, i, j, 0)),
                pl.BlockSpec((c, Cout), lambda b, i, j: (0, 0)),
            ],
            out_specs=pl.BlockSpec((1, Hh, Wh, Cout),
                                   lambda b, i, j: (b, i, j, 0)),
        ),
        compiler_params=pltpu.CompilerParams(
            dimension_semantics=("parallel", "parallel", "parallel")),
    )(y, fu, xs_small, l2g, w.astype(jnp.bfloat16))
    return out


def _dft_mats(Hh, Ww, Vp):
    import numpy as np
    V = Ww // 2 + 1
    u = np.arange(Hh)
    th = 2.0 * np.pi * np.outer(u, u) / Hh
    fhr = np.cos(th) / np.sqrt(Hh)
    fhi = -np.sin(th) / np.sqrt(Hh)
    w = np.arange(Ww)
    v = np.arange(Vp)
    ph = 2.0 * np.pi * np.outer(w, v) / Ww
    mask = (v < V).astype(np.float64)
    fwr = np.cos(ph) / np.sqrt(Ww) * mask
    fwi = -np.sin(ph) / np.sqrt(Ww) * mask
    ihr = np.cos(th) / np.sqrt(Hh)
    ihi = np.sin(th) / np.sqrt(Hh)
    alpha = np.where((v == 0) | (v == Ww // 2), 1.0, 2.0) * mask
    pw = 2.0 * np.pi * np.outer(v, w) / Ww
    icw = alpha[:, None] * np.cos(pw) / np.sqrt(Ww)
    isw = -alpha[:, None] * np.sin(pw) / np.sqrt(Ww)
    return [jnp.asarray(m, jnp.bfloat16)
            for m in (fhr, fhi, fwr, fwi, ihr, ihi, icw, isw)]


def _bspec(shape, blocked_dims=1):
    if blocked_dims == 0:
        return pl.BlockSpec(shape, lambda b: tuple(0 for _ in shape))
    return pl.BlockSpec((1,) + shape[1:],
                        lambda b: (b,) + tuple(0 for _ in shape[1:]))


def _batch_call(body, ins, blocked, out_shapes, out_dtype=jnp.bfloat16):
    B = ins[0].shape[0]
    return pl.pallas_call(
        body,
        out_shape=tuple(jax.ShapeDtypeStruct(s, out_dtype)
                        for s in out_shapes),
        grid_spec=pltpu.PrefetchScalarGridSpec(
            num_scalar_prefetch=0,
            grid=(B,),
            in_specs=[_bspec(a.shape, d) for a, d in zip(ins, blocked)],
            out_specs=[_bspec(s) for s in out_shapes],
        ),
        compiler_params=pltpu.CompilerParams(
            dimension_semantics=("parallel",)),
    )(*ins)


_BF = jnp.bfloat16
_F32 = jnp.float32


def _mxt(x_bf, i_ref):
    return lax.dot_general(i_ref[...], x_bf, (((1,), (1,)), ((), ())),
                           preferred_element_type=_F32)


def _make_pw2_body(HW, cg):
    def body(x_ref, w_ref, sb_ref, ic_ref, y_ref, yt_ref):
        x = x_ref[0].reshape(HW, cg).astype(_BF)
        y = jnp.dot(x, w_ref[...], preferred_element_type=_F32)
        y = jnp.maximum(y * sb_ref[0:1] + sb_ref[1:2], 0.0)
        yb = y.astype(_BF)
        y_ref[0] = yb
        yt_ref[0] = _mxt(yb, ic_ref).astype(_BF)
    return body


def _fwdw_body(yt_ref, tt_ref, fwr, fwi, gwr, gwi, iv, iv2,
               crt_ref, cit_ref, lrt_ref, lit_ref):
    yt = yt_ref[0]
    zr = jnp.dot(yt, fwr[...], preferred_element_type=_F32)
    zi = jnp.dot(yt, fwi[...], preferred_element_type=_F32)
    crt_ref[0] = _mxt(zr.astype(_BF), iv).astype(_BF)
    cit_ref[0] = _mxt(zi.astype(_BF), iv).astype(_BF)
    tt = tt_ref[0]
    wr = jnp.dot(tt, gwr[...], preferred_element_type=_F32)
    wi = jnp.dot(tt, gwi[...], preferred_element_type=_F32)
    lrt_ref[0] = _mxt(wr.astype(_BF), iv2).astype(_BF)
    lit_ref[0] = _mxt(wi.astype(_BF), iv2).astype(_BF)


def _fwdh_body(cr_ref, ci_ref, lr_ref, li_ref, fhr, fhi, ghr, ghi, iu, iu2,
               fr_ref, fi_ref, gr_ref, gi_ref):
    cr = cr_ref[0]
    ci = ci_ref[0]
    fr = (jnp.dot(cr, fhr[...], preferred_element_type=_F32)
          - jnp.dot(ci, fhi[...], preferred_element_type=_F32))
    fi = (jnp.dot(cr, fhi[...], preferred_element_type=_F32)
          + jnp.dot(ci, fhr[...], preferred_element_type=_F32))
    fr_ref[0] = _mxt(fr.astype(_BF), iu).astype(_BF)
    fi_ref[0] = _mxt(fi.astype(_BF), iu).astype(_BF)
    lr = lr_ref[0]
    li = li_ref[0]
    gr = (jnp.dot(lr, ghr[...], preferred_element_type=_F32)
          - jnp.dot(li, ghi[...], preferred_element_type=_F32))
    gi = (jnp.dot(lr, ghi[...], preferred_element_type=_F32)
          + jnp.dot(li, ghr[...], preferred_element_type=_F32))
    gr_ref[0] = _mxt(gr.astype(_BF), iu2).astype(_BF)
    gi_ref[0] = _mxt(gi.astype(_BF), iu2).astype(_BF)


def _make_freqconv_body(cc):
    def body(fr_ref, fi_ref, lr_ref, li_ref, wfu, wlfu, sbf, sbl,
             gr_ref, gi_ref, hr_ref, hi_ref):
        fr = fr_ref[0]
        fi = fi_ref[0]
        g = (jnp.dot(fr, wfu[:cc], preferred_element_type=_F32)
             + jnp.dot(fi, wfu[cc:], preferred_element_type=_F32))
        g = jnp.maximum(g * sbf[0:1] + sbf[1:2], 0.0)
        gr_ref[0] = g[:, :cc].astype(_BF)
        gi_ref[0] = g[:, cc:].astype(_BF)
        lr = lr_ref[0]
        li = li_ref[0]
        h = (jnp.dot(lr, wlfu[:cc], preferred_element_type=_F32)
             + jnp.dot(li, wlfu[cc:], preferred_element_type=_F32))
        h = jnp.maximum(h * sbl[0:1] + sbl[1:2], 0.0)
        hr_ref[0] = h[:, :cc].astype(_BF)
        hi_ref[0] = h[:, cc:].astype(_BF)
    return body


def _mxt2(x_bf, i_ref):
    return lax.dot_general(x_bf, i_ref[...], (((0,), (0,)), ((), ())),
                           preferred_element_type=_F32)


def _invh_body(gr_ref, gi_ref, hr_ref, hi_ref, ihr, ihi, jhr, jhi, iu, iu2,
               dr_ref, di_ref, er_ref, ei_ref):
    gr = gr_ref[0]
    gi = gi_ref[0]
    dr = (jnp.dot(ihr[...], gr, preferred_element_type=_F32)
          - jnp.dot(ihi[...], gi, preferred_element_type=_F32))
    di = (jnp.dot(ihr[...], gi, preferred_element_type=_F32)
          + jnp.dot(ihi[...], gr, preferred_element_type=_F32))
    dr_ref[0] = _mxt2(dr.astype(_BF), iu).astype(_BF)
    di_ref[0] = _mxt2(di.astype(_BF), iu).astype(_BF)
    hr = hr_ref[0]
    hi = hi_ref[0]
    er = (jnp.dot(jhr[...], hr, preferred_element_type=_F32)
          - jnp.dot(jhi[...], hi, preferred_element_type=_F32))
    ei = (jnp.dot(jhr[...], hi, preferred_element_type=_F32)
          + jnp.dot(jhi[...], hr, preferred_element_type=_F32))
    er_ref[0] = _mxt2(er.astype(_BF), iu2).astype(_BF)
    ei_ref[0] = _mxt2(ei.astype(_BF), iu2).astype(_BF)


def _invw_body(dr_ref, di_ref, ldr_ref, ldi_ref, icw, isw, jcw, jsw,
               iw, iw2, fu_ref, xs_ref):
    dr = dr_ref[0]
    di = di_ref[0]
    fu = (lax.dot_general(icw[...], dr, (((0,), (0,)), ((), ())),
                          preferred_element_type=_F32)
          + lax.dot_general(isw[...], di, (((0,), (0,)), ((), ())),
                            preferred_element_type=_F32))
    fu_ref[0] = _mxt2(fu.astype(_BF), iw).astype(_BF)
    ldr = ldr_ref[0]
    ldi = ldi_ref[0]
    xs = (lax.dot_general(jcw[...], ldr, (((0,), (0,)), ((), ())),
                          preferred_element_type=_F32)
          + lax.dot_general(jsw[...], ldi, (((0,), (0,)), ((), ())),
                            preferred_element_type=_F32))
    xs_ref[0] = _mxt2(xs.astype(_BF), iw2).astype(_BF)


def _spectral_pipeline(x_g, l2g, w1, w_fu, w_lfu, w2, sb1, sbfu, sblfu):
    B, H, W, cg = x_g.shape
    c = w1.shape[-1]
    H2, W2 = H // 2, W // 2
    c4 = c // 4
    Vp = ((W // 2 + 1) + 7) // 8 * 8
    Vp2 = ((W2 // 2 + 1) + 7) // 8 * 8
    fhr, fhi, fwr, fwi, ihr, ihi, icw, isw = _dft_mats(H, W, Vp)
    ghr, ghi, gwr, gwi, jhr, jhi, jcw, jsw = _dft_mats(H2, W2, Vp2)
    ic = jnp.eye(c, dtype=_BF)
    iv, iv2 = jnp.eye(Vp, dtype=_BF), jnp.eye(Vp2, dtype=_BF)
    iu, iu2 = jnp.eye(H, dtype=_BF), jnp.eye(H2, dtype=_BF)

    y, yt = _batch_call(
        _make_pw2_body(H * W, cg),
        [x_g, w1.astype(_BF), sb1, ic],
        [1, 0, 0, 0],
        [(B, H * W, c), (B, c, H * W)])

    yt4 = yt.reshape(B, c, H, W)
    tt = jnp.concatenate(
        [yt4[:, :c4, :H2, :W2], yt4[:, :c4, H2:, :W2],
         yt4[:, :c4, :H2, W2:], yt4[:, :c4, H2:, W2:]], axis=1)

    crt, cit, lrt, lit = _batch_call(
        _fwdw_body,
        [yt4.reshape(B, c * H, W), tt.reshape(B, c * H2, W2),
         fwr, fwi, gwr, gwi, iv, iv2],
        [1, 1, 0, 0, 0, 0, 0, 0],
        [(B, Vp, c * H), (B, Vp, c * H),
         (B, Vp2, c * H2), (B, Vp2, c * H2)])

    rs1 = lambda a, vp, hh: a.reshape(B, vp * c, hh)
    fr, fi, glr, gli = _batch_call(
        _fwdh_body,
        [rs1(crt, Vp, H), rs1(cit, Vp, H), rs1(lrt, Vp2, H2),
         rs1(lit, Vp2, H2), fhr, fhi, ghr, ghi, iu, iu2],
        [1, 1, 1, 1, 0, 0, 0, 0, 0, 0],
        [(B, H, Vp * c), (B, H, Vp * c),
         (B, H2, Vp2 * c), (B, H2, Vp2 * c)])

    rs2 = lambda a, hh, vp: a.reshape(B, hh * vp, c)
    gr, gi, hr, hi = _batch_call(
        _make_freqconv_body(c),
        [rs2(fr, H, Vp), rs2(fi, H, Vp), rs2(glr, H2, Vp2),
         rs2(gli, H2, Vp2), w_fu.astype(_BF), w_lfu.astype(_BF),
         sbfu, sblfu],
        [1, 1, 1, 1, 0, 0, 0, 0],
        [(B, H * Vp, c), (B, H * Vp, c),
         (B, H2 * Vp2, c), (B, H2 * Vp2, c)])

    iw, iw2 = jnp.eye(W, dtype=_BF), jnp.eye(W2, dtype=_BF)
    rs3 = lambda a, hh, vp: a.reshape(B, hh, vp * c)
    dr, di, ldr, ldi = _batch_call(
        _invh_body,
        [rs3(gr, H, Vp), rs3(gi, H, Vp), rs3(hr, H2, Vp2),
         rs3(hi, H2, Vp2), ihr, ihi, jhr, jhi, iu, iu2],
        [1, 1, 1, 1, 0, 0, 0, 0, 0, 0],
        [(B, Vp * c, H), (B, Vp * c, H),
         (B, Vp2 * c, H2), (B, Vp2 * c, H2)])

    rs4 = lambda a, vp, hh: a.reshape(B, vp, c * hh)
    fu_t, xs_t = _batch_call(
        _invw_body,
        [rs4(dr, Vp, H), rs4(di, Vp, H), rs4(ldr, Vp2, H2),
         rs4(ldi, Vp2, H2), icw, isw, jcw, jsw, iw, iw2],
        [1, 1, 1, 1, 0, 0, 0, 0, 0, 0],
        [(B, c * H, W), (B, c * H2, W2)])

    fu = jnp.transpose(fu_t.reshape(B, c, H, W), (0, 2, 3, 1))
    xs = jnp.transpose(xs_t.reshape(B, c, H2, W2), (0, 2, 3, 1))
    return y, fu, xs


def _lfu_fold(y):
    B, H, W, c = y.shape
    c4 = c // 4
    t = y[..., :c4]
    t = jnp.concatenate([t[:, : H // 2], t[:, H // 2:]], axis=-1)
    t = jnp.concatenate([t[:, :, : W // 2], t[:, :, W // 2:]], axis=-1)
    return t


def _fourier_unit(t, w, gamma, beta, mean, var):
    Hh, Ww, cch = t.shape[1], t.shape[2], t.shape[3]
    f = jnp.fft.rfft2(t, axes=(1, 2), norm="ortho")
    fr = jnp.concatenate([f.real, f.imag], axis=-1).astype(jnp.float32)
    s, b = _bn_scale_bias(gamma, beta, mean, var)
    g = _pw_affine_relu(fr, w, s, b)
    gc = lax.complex(g[..., :cch], g[..., cch:])
    return jnp.fft.irfft2(gc, s=(Hh, Ww), axes=(1, 2),
                          norm="ortho").astype(jnp.float32)


def kernel(x_l, x_g, w_l2l, w_g2l, w_l2g, w1, w_fu, w_lfu, w2,
           bn1_gamma, bn1_beta, bn1_mean, bn1_var,
           fu_bn_gamma, fu_bn_beta, fu_bn_mean, fu_bn_var,
           lfu_bn_gamma, lfu_bn_beta, lfu_bn_mean, lfu_bn_var):
    B, H, W, cl = x_l.shape
    cg = x_g.shape[-1]
    ocl = w_l2l.shape[-1]
    ocg = w_l2g.shape[-1]
    C = cl + cg

    wc = jnp.zeros((3, 3, C, ocl + ocg), jnp.float32)
    wc = wc.at[:, :, :cl, :ocl].set(w_l2l)
    wc = wc.at[:, :, cl:, :ocl].set(w_g2l)
    wc = wc.at[:, :, :cl, ocl:].set(w_l2g)
    wc = wc.reshape(9 * C, ocl + ocg).astype(jnp.bfloat16)

    out_l, l2g = _conv3x3_dual(x_l, x_g, wc, ocl, ocg)

    s1, b1 = _bn_scale_bias(bn1_gamma, bn1_beta, bn1_mean, bn1_var)
    sfu, bfu = _bn_scale_bias(fu_bn_gamma, fu_bn_beta, fu_bn_mean, fu_bn_var)
    slf, blf = _bn_scale_bias(lfu_bn_gamma, lfu_bn_beta, lfu_bn_mean, lfu_bn_var)
    sb1 = jnp.stack([s1, b1]).astype(jnp.float32)
    sbfu = jnp.stack([sfu, bfu]).astype(jnp.float32)
    sblfu = jnp.stack([slf, blf]).astype(jnp.float32)

    y, fu, xs = _spectral_pipeline(x_g, l2g, w1, w_fu, w_lfu, w2,
                                   sb1, sbfu, sblfu)
    out_g = _conv2_fused(y.reshape(x_g.shape[:3] + (w1.shape[-1],)),
                         fu, xs, l2g, w2)
    return out_l, out_g


# speedup vs baseline: 1.5634x; 1.0158x over previous
import math

import jax
import jax.numpy as jnp
from jax import lax
from jax.experimental import pallas as pl
from jax.experimental.pallas import tpu as pltpu


def _bn_scale_bias(gamma, beta, mean, var, eps=1e-5):
    s = gamma / jnp.sqrt(var + eps)
    return s, beta - mean * s


def _make_conv_body(th, W, cl, cg, ocl):
    C = cl + cg
    bf = jnp.bfloat16

    def body(xl_ref, xg_ref, tl_ref, tg_ref, bl_ref, bg_ref, w_ref,
             outl_ref, l2g_ref, xp_ref, col_ref):
        i = pl.program_id(1)
        n = pl.num_programs(1)

        xp_ref[1:th + 1, 1:W + 1, :cl] = xl_ref[0].astype(bf)
        xp_ref[1:th + 1, 1:W + 1, cl:] = xg_ref[0].astype(bf)

        @pl.when(i == 0)
        def _():
            xp_ref[0:1, 1:W + 1, :cl] = xl_ref[0, 1:2].astype(bf)
            xp_ref[0:1, 1:W + 1, cl:] = xg_ref[0, 1:2].astype(bf)

        @pl.when(i > 0)
        def _():
            xp_ref[0:1, 1:W + 1, :cl] = tl_ref[0, 7:8].astype(bf)
            xp_ref[0:1, 1:W + 1, cl:] = tg_ref[0, 7:8].astype(bf)

        @pl.when(i == n - 1)
        def _():
            xp_ref[th + 1:th + 2, 1:W + 1, :cl] = xl_ref[0, th - 2:th - 1].astype(bf)
            xp_ref[th + 1:th + 2, 1:W + 1, cl:] = xg_ref[0, th - 2:th - 1].astype(bf)

        @pl.when(i < n - 1)
        def _():
            xp_ref[th + 1:th + 2, 1:W + 1, :cl] = bl_ref[0, 0:1].astype(bf)
            xp_ref[th + 1:th + 2, 1:W + 1, cl:] = bg_ref[0, 0:1].astype(bf)

        xp_ref[:, 0:1, :] = xp_ref[:, 2:3, :]
        xp_ref[:, W + 1:W + 2, :] = xp_ref[:, W - 1:W, :]

        for dy in range(3):
            for dx in range(3):
                t = dy * 3 + dx
                col_ref[:, t * C:(t + 1) * C] = (
                    xp_ref[dy:dy + th, dx:dx + W, :].reshape(th * W, C))

        y = jnp.dot(col_ref[...], w_ref[...],
                    preferred_element_type=jnp.float32)
        outl_ref[0] = y[:, :ocl]
        l2g_ref[0] = y[:, ocl:].astype(bf)

    return body


def _conv3x3_dual(x_l, x_g, wc, ocl, ocg, th=32):
    B, H, W, cl = x_l.shape
    cg = x_g.shape[-1]
    C = cl + cg
    n_th = H // th
    thb = th // 8

    outl, l2g = pl.pallas_call(
        _make_conv_body(th, W, cl, cg, ocl),
        out_shape=(jax.ShapeDtypeStruct((B, H * W, ocl), jnp.float32),
                   jax.ShapeDtypeStruct((B, H * W, ocg), jnp.bfloat16)),
        grid_spec=pltpu.PrefetchScalarGridSpec(
            num_scalar_prefetch=0,
            grid=(B, n_th),
            in_specs=[
                pl.BlockSpec((1, th, W, cl), lambda b, i: (b, i, 0, 0)),
                pl.BlockSpec((1, th, W, cg), lambda b, i: (b, i, 0, 0)),
                pl.BlockSpec((1, 8, W, cl),
                             lambda b, i: (b, jnp.maximum(i * thb - 1, 0), 0, 0)),
                pl.BlockSpec((1, 8, W, cg),
                             lambda b, i: (b, jnp.maximum(i * thb - 1, 0), 0, 0)),
                pl.BlockSpec((1, 8, W, cl),
                             lambda b, i: (b, jnp.minimum((i + 1) * thb,
                                                          H // 8 - 1), 0, 0)),
                pl.BlockSpec((1, 8, W, cg),
                             lambda b, i: (b, jnp.minimum((i + 1) * thb,
                                                          H // 8 - 1), 0, 0)),
                pl.BlockSpec((9 * C, ocl + ocg), lambda b, i: (0, 0)),
            ],
            out_specs=[
                pl.BlockSpec((1, th * W, ocl), lambda b, i: (b, i, 0)),
                pl.BlockSpec((1, th * W, ocg), lambda b, i: (b, i, 0)),
            ],
            scratch_shapes=[
                pltpu.VMEM((th + 2, W + 2, C), jnp.bfloat16),
                pltpu.VMEM((th * W, 9 * C), jnp.bfloat16),
            ],
        ),
        compiler_params=pltpu.CompilerParams(
            dimension_semantics=("parallel", "parallel"),
            vmem_limit_bytes=96 << 20),
    )(x_l, x_g, x_l, x_g, x_l, x_g, wc)
    return (outl.reshape(B, H, W, ocl), l2g.reshape(B, H, W, ocg))


def _pw_body(x_ref, w_ref, sb_ref, out_ref):
    y = jnp.dot(x_ref[...].astype(jnp.bfloat16), w_ref[...],
                preferred_element_type=jnp.float32)
    y = y * sb_ref[0:1] + sb_ref[1:2]
    out_ref[...] = jnp.maximum(y, 0.0)


def _pw_affine_relu(x, w, scale, bias, tm=1024):
    lead = x.shape[:-1]
    Cin = x.shape[-1]
    Cout = w.shape[-1]
    M = int(math.prod(lead))
    grid = -(-M // tm)
    sb = jnp.stack([scale, bias]).astype(jnp.float32)
    out = pl.pallas_call(
        _pw_body,
        out_shape=jax.ShapeDtypeStruct((M, Cout), jnp.float32),
        grid_spec=pltpu.PrefetchScalarGridSpec(
            num_scalar_prefetch=0,
            grid=(grid,),
            in_specs=[
                pl.BlockSpec((tm, Cin), lambda i: (i, 0)),
                pl.BlockSpec((Cin, Cout), lambda i: (0, 0)),
                pl.BlockSpec((2, Cout), lambda i: (0, 0)),
            ],
            out_specs=pl.BlockSpec((tm, Cout), lambda i: (i, 0)),
        ),
        compiler_params=pltpu.CompilerParams(
            dimension_semantics=("parallel",)),
    )(x.reshape(M, Cin), w.astype(jnp.bfloat16), sb)
    return out.reshape(lead + (Cout,))


def _conv2_body(y_ref, fu_ref, xs_ref, l2g_ref, w_ref, out_ref):
    s = (y_ref[0].astype(jnp.float32) + fu_ref[0].astype(jnp.float32)
         + xs_ref[0].astype(jnp.float32))
    Hh, Wh, c = s.shape
    o = jnp.dot(s.reshape(Hh * Wh, c).astype(jnp.bfloat16), w_ref[...],
                preferred_element_type=jnp.float32)
    o = o + l2g_ref[0].reshape(Hh * Wh, o.shape[-1])
    out_ref[0] = o.reshape(Hh, Wh, o.shape[-1])


def _conv2_fused(y, fu, xs_small, l2g, w):
    B, H, W, c = y.shape
    Cout = w.shape[-1]
    Hh, Wh = H // 2, W // 2
    out = pl.pallas_call(
        _conv2_body,
        out_shape=jax.ShapeDtypeStruct((B, H, W, Cout), jnp.float32),
        grid_spec=pltpu.PrefetchScalarGridSpec(
            num_scalar_prefetch=0,
            grid=(B, 2, 2),
            in_specs=[
                pl.BlockSpec((1, Hh, Wh, c), lambda b, i, j: (b, i, j, 0)),
                pl.BlockSpec((1, Hh, Wh, c), lambda b, i, j: (b, i, j, 0)),
                pl.BlockSpec((1, Hh, Wh, c), lambda b, i, j: (b, 0, 0, 0)),
                pl.BlockSpec((1, Hh, Wh, Cout), lambda b, i, j: (b, i, j, 0)),
                pl.BlockSpec((c, Cout), lambda b, i, j: (0, 0)),
            ],
            out_specs=pl.BlockSpec((1, Hh, Wh, Cout),
                                   lambda b, i, j: (b, i, j, 0)),
        ),
        compiler_params=pltpu.CompilerParams(
            dimension_semantics=("parallel", "parallel", "parallel")),
    )(y, fu, xs_small, l2g, w.astype(jnp.bfloat16))
    return out


def _dft_mats(Hh, Ww, Vp):
    import numpy as np
    V = Ww // 2 + 1
    u = np.arange(Hh)
    th = 2.0 * np.pi * np.outer(u, u) / Hh
    fhr = np.cos(th) / np.sqrt(Hh)
    fhi = -np.sin(th) / np.sqrt(Hh)
    w = np.arange(Ww)
    v = np.arange(Vp)
    ph = 2.0 * np.pi * np.outer(w, v) / Ww
    mask = (v < V).astype(np.float64)
    fwr = np.cos(ph) / np.sqrt(Ww) * mask
    fwi = -np.sin(ph) / np.sqrt(Ww) * mask
    ihr = np.cos(th) / np.sqrt(Hh)
    ihi = np.sin(th) / np.sqrt(Hh)
    alpha = np.where((v == 0) | (v == Ww // 2), 1.0, 2.0) * mask
    pw = 2.0 * np.pi * np.outer(v, w) / Ww
    icw = alpha[:, None] * np.cos(pw) / np.sqrt(Ww)
    isw = -alpha[:, None] * np.sin(pw) / np.sqrt(Ww)
    return [jnp.asarray(m, jnp.bfloat16)
            for m in (fhr, fhi, fwr, fwi, ihr, ihi, icw, isw)]


def _bspec(shape, blocked_dims=1):
    if blocked_dims == 0:
        return pl.BlockSpec(shape, lambda b: tuple(0 for _ in shape))
    return pl.BlockSpec((1,) + shape[1:],
                        lambda b: (b,) + tuple(0 for _ in shape[1:]))


def _batch_call(body, ins, blocked, out_shapes, out_dtype=jnp.bfloat16):
    B = ins[0].shape[0]
    return pl.pallas_call(
        body,
        out_shape=tuple(jax.ShapeDtypeStruct(s, out_dtype)
                        for s in out_shapes),
        grid_spec=pltpu.PrefetchScalarGridSpec(
            num_scalar_prefetch=0,
            grid=(B,),
            in_specs=[_bspec(a.shape, d) for a, d in zip(ins, blocked)],
            out_specs=[_bspec(s) for s in out_shapes],
        ),
        compiler_params=pltpu.CompilerParams(
            dimension_semantics=("parallel",)),
    )(*ins)


_BF = jnp.bfloat16
_F32 = jnp.float32


def _mxt(x_bf, i_ref):
    return lax.dot_general(i_ref[...], x_bf, (((1,), (1,)), ((), ())),
                           preferred_element_type=_F32)


def _make_pw2_body(HW, cg):
    def body(x_ref, w_ref, sb_ref, ic_ref, y_ref, yt_ref):
        x = x_ref[0].reshape(HW, cg).astype(_BF)
        y = jnp.dot(x, w_ref[...], preferred_element_type=_F32)
        y = jnp.maximum(y * sb_ref[0:1] + sb_ref[1:2], 0.0)
        yb = y.astype(_BF)
        y_ref[0] = yb
        yt_ref[0] = _mxt(yb, ic_ref).astype(_BF)
    return body


def _fwdw_body(yt_ref, tt_ref, fwr, fwi, gwr, gwi, iv, iv2,
               crt_ref, cit_ref, lrt_ref, lit_ref):
    yt = yt_ref[0]
    zr = jnp.dot(yt, fwr[...], preferred_element_type=_F32)
    zi = jnp.dot(yt, fwi[...], preferred_element_type=_F32)
    crt_ref[0] = _mxt(zr.astype(_BF), iv).astype(_BF)
    cit_ref[0] = _mxt(zi.astype(_BF), iv).astype(_BF)
    tt = tt_ref[0]
    wr = jnp.dot(tt, gwr[...], preferred_element_type=_F32)
    wi = jnp.dot(tt, gwi[...], preferred_element_type=_F32)
    lrt_ref[0] = _mxt(wr.astype(_BF), iv2).astype(_BF)
    lit_ref[0] = _mxt(wi.astype(_BF), iv2).astype(_BF)


def _fwdh_body(cr_ref, ci_ref, lr_ref, li_ref, fhr, fhi, ghr, ghi, iu, iu2,
               fr_ref, fi_ref, gr_ref, gi_ref):
    cr = cr_ref[0]
    ci = ci_ref[0]
    fr = (jnp.dot(cr, fhr[...], preferred_element_type=_F32)
          - jnp.dot(ci, fhi[...], preferred_element_type=_F32))
    fi = (jnp.dot(cr, fhi[...], preferred_element_type=_F32)
          + jnp.dot(ci, fhr[...], preferred_element_type=_F32))
    fr_ref[0] = _mxt(fr.astype(_BF), iu).astype(_BF)
    fi_ref[0] = _mxt(fi.astype(_BF), iu).astype(_BF)
    lr = lr_ref[0]
    li = li_ref[0]
    gr = (jnp.dot(lr, ghr[...], preferred_element_type=_F32)
          - jnp.dot(li, ghi[...], preferred_element_type=_F32))
    gi = (jnp.dot(lr, ghi[...], preferred_element_type=_F32)
          + jnp.dot(li, ghr[...], preferred_element_type=_F32))
    gr_ref[0] = _mxt(gr.astype(_BF), iu2).astype(_BF)
    gi_ref[0] = _mxt(gi.astype(_BF), iu2).astype(_BF)


def _make_freqconv_body(cc):
    def body(fr_ref, fi_ref, lr_ref, li_ref, wfu, wlfu, sbf, sbl,
             gr_ref, gi_ref, hr_ref, hi_ref):
        fr = fr_ref[0]
        fi = fi_ref[0]
        g = (jnp.dot(fr, wfu[:cc], preferred_element_type=_F32)
             + jnp.dot(fi, wfu[cc:], preferred_element_type=_F32))
        g = jnp.maximum(g * sbf[0:1] + sbf[1:2], 0.0)
        gr_ref[0] = g[:, :cc].astype(_BF)
        gi_ref[0] = g[:, cc:].astype(_BF)
        lr = lr_ref[0]
        li = li_ref[0]
        h = (jnp.dot(lr, wlfu[:cc], preferred_element_type=_F32)
             + jnp.dot(li, wlfu[cc:], preferred_element_type=_F32))
        h = jnp.maximum(h * sbl[0:1] + sbl[1:2], 0.0)
        hr_ref[0] = h[:, :cc].astype(_BF)
        hi_ref[0] = h[:, cc:].astype(_BF)
    return body


def _mxt2(x_bf, i_ref):
    return lax.dot_general(x_bf, i_ref[...], (((0,), (0,)), ((), ())),
                           preferred_element_type=_F32)


def _invh_body(gr_ref, gi_ref, hr_ref, hi_ref, ihr, ihi, jhr, jhi, iu, iu2,
               dr_ref, di_ref, er_ref, ei_ref):
    gr = gr_ref[0]
    gi = gi_ref[0]
    dr = (jnp.dot(ihr[...], gr, preferred_element_type=_F32)
          - jnp.dot(ihi[...], gi, preferred_element_type=_F32))
    di = (jnp.dot(ihr[...], gi, preferred_element_type=_F32)
          + jnp.dot(ihi[...], gr, preferred_element_type=_F32))
    dr_ref[0] = _mxt2(dr.astype(_BF), iu).astype(_BF)
    di_ref[0] = _mxt2(di.astype(_BF), iu).astype(_BF)
    hr = hr_ref[0]
    hi = hi_ref[0]
    er = (jnp.dot(jhr[...], hr, preferred_element_type=_F32)
          - jnp.dot(jhi[...], hi, preferred_element_type=_F32))
    ei = (jnp.dot(jhr[...], hi, preferred_element_type=_F32)
          + jnp.dot(jhi[...], hr, preferred_element_type=_F32))
    er_ref[0] = _mxt2(er.astype(_BF), iu2).astype(_BF)
    ei_ref[0] = _mxt2(ei.astype(_BF), iu2).astype(_BF)


def _invw_body(dr_ref, di_ref, ldr_ref, ldi_ref, icw, isw, jcw, jsw,
               iw, iw2, fu_ref, xs_ref):
    dr = dr_ref[0]
    di = di_ref[0]
    fu = (lax.dot_general(icw[...], dr, (((0,), (0,)), ((), ())),
                          preferred_element_type=_F32)
          + lax.dot_general(isw[...], di, (((0,), (0,)), ((), ())),
                            preferred_element_type=_F32))
    fu_ref[0] = _mxt2(fu.astype(_BF), iw).astype(_BF)
    ldr = ldr_ref[0]
    ldi = ldi_ref[0]
    xs = (lax.dot_general(jcw[...], ldr, (((0,), (0,)), ((), ())),
                          preferred_element_type=_F32)
          + lax.dot_general(jsw[...], ldi, (((0,), (0,)), ((), ())),
                            preferred_element_type=_F32))
    xs_ref[0] = _mxt2(xs.astype(_BF), iw2).astype(_BF)


def _spectral_pipeline(x_g, l2g, w1, w_fu, w_lfu, w2, sb1, sbfu, sblfu):
    B, H, W, cg = x_g.shape
    c = w1.shape[-1]
    H2, W2 = H // 2, W // 2
    c4 = c // 4
    Vp = ((W // 2 + 1) + 7) // 8 * 8
    Vp2 = ((W2 // 2 + 1) + 7) // 8 * 8
    fhr, fhi, fwr, fwi, ihr, ihi, icw, isw = _dft_mats(H, W, Vp)
    ghr, ghi, gwr, gwi, jhr, jhi, jcw, jsw = _dft_mats(H2, W2, Vp2)
    ic = jnp.eye(c, dtype=_BF)
    iv, iv2 = jnp.eye(Vp, dtype=_BF), jnp.eye(Vp2, dtype=_BF)
    iu, iu2 = jnp.eye(H, dtype=_BF), jnp.eye(H2, dtype=_BF)

    y, yt = _batch_call(
        _make_pw2_body(H * W, cg),
        [x_g, w1.astype(_BF), sb1, ic],
        [1, 0, 0, 0],
        [(B, H * W, c), (B, c, H * W)])

    yt4 = yt.reshape(B, c, H, W)
    tt = jnp.concatenate(
        [yt4[:, :c4, :H2, :W2], yt4[:, :c4, H2:, :W2],
         yt4[:, :c4, :H2, W2:], yt4[:, :c4, H2:, W2:]], axis=1)

    crt, cit, lrt, lit = _batch_call(
        _fwdw_body,
        [yt4.reshape(B, c * H, W), tt.reshape(B, c * H2, W2),
         fwr, fwi, gwr, gwi, iv, iv2],
        [1, 1, 0, 0, 0, 0, 0, 0],
        [(B, Vp, c * H), (B, Vp, c * H),
         (B, Vp2, c * H2), (B, Vp2, c * H2)])

    rs1 = lambda a, vp, hh: a.reshape(B, vp * c, hh)
    fr, fi, glr, gli = _batch_call(
        _fwdh_body,
        [rs1(crt, Vp, H), rs1(cit, Vp, H), rs1(lrt, Vp2, H2),
         rs1(lit, Vp2, H2), fhr, fhi, ghr, ghi, iu, iu2],
        [1, 1, 1, 1, 0, 0, 0, 0, 0, 0],
        [(B, H, Vp * c), (B, H, Vp * c),
         (B, H2, Vp2 * c), (B, H2, Vp2 * c)])

    rs2 = lambda a, hh, vp: a.reshape(B, hh * vp, c)
    gr, gi, hr, hi = _batch_call(
        _make_freqconv_body(c),
        [rs2(fr, H, Vp), rs2(fi, H, Vp), rs2(glr, H2, Vp2),
         rs2(gli, H2, Vp2), w_fu.astype(_BF), w_lfu.astype(_BF),
         sbfu, sblfu],
        [1, 1, 1, 1, 0, 0, 0, 0],
        [(B, H * Vp, c), (B, H * Vp, c),
         (B, H2 * Vp2, c), (B, H2 * Vp2, c)])

    iw, iw2 = jnp.eye(W, dtype=_BF), jnp.eye(W2, dtype=_BF)
    rs3 = lambda a, hh, vp: a.reshape(B, hh, vp * c)
    dr, di, ldr, ldi = _batch_call(
        _invh_body,
        [rs3(gr, H, Vp), rs3(gi, H, Vp), rs3(hr, H2, Vp2),
         rs3(hi, H2, Vp2), ihr, ihi, jhr, jhi, iu, iu2],
        [1, 1, 1, 1, 0, 0, 0, 0, 0, 0],
        [(B, Vp * c, H), (B, Vp * c, H),
         (B, Vp2 * c, H2), (B, Vp2 * c, H2)])

    rs4 = lambda a, vp, hh: a.reshape(B, vp, c * hh)
    fu_t, xs_t = _batch_call(
        _invw_body,
        [rs4(dr, Vp, H), rs4(di, Vp, H), rs4(ldr, Vp2, H2),
         rs4(ldi, Vp2, H2), icw, isw, jcw, jsw, iw, iw2],
        [1, 1, 1, 1, 0, 0, 0, 0, 0, 0],
        [(B, c * H, W), (B, c * H2, W2)])

    fu = jnp.transpose(fu_t.reshape(B, c, H, W), (0, 2, 3, 1))
    xs = jnp.transpose(xs_t.reshape(B, c, H2, W2), (0, 2, 3, 1))
    return y, fu, xs


def _lfu_fold(y):
    B, H, W, c = y.shape
    c4 = c // 4
    t = y[..., :c4]
    t = jnp.concatenate([t[:, : H // 2], t[:, H // 2:]], axis=-1)
    t = jnp.concatenate([t[:, :, : W // 2], t[:, :, W // 2:]], axis=-1)
    return t


def _fourier_unit(t, w, gamma, beta, mean, var):
    Hh, Ww, cch = t.shape[1], t.shape[2], t.shape[3]
    f = jnp.fft.rfft2(t, axes=(1, 2), norm="ortho")
    fr = jnp.concatenate([f.real, f.imag], axis=-1).astype(jnp.float32)
    s, b = _bn_scale_bias(gamma, beta, mean, var)
    g = _pw_affine_relu(fr, w, s, b)
    gc = lax.complex(g[..., :cch], g[..., cch:])
    return jnp.fft.irfft2(gc, s=(Hh, Ww), axes=(1, 2),
                          norm="ortho").astype(jnp.float32)


def kernel(x_l, x_g, w_l2l, w_g2l, w_l2g, w1, w_fu, w_lfu, w2,
           bn1_gamma, bn1_beta, bn1_mean, bn1_var,
           fu_bn_gamma, fu_bn_beta, fu_bn_mean, fu_bn_var,
           lfu_bn_gamma, lfu_bn_beta, lfu_bn_mean, lfu_bn_var):
    B, H, W, cl = x_l.shape
    cg = x_g.shape[-1]
    ocl = w_l2l.shape[-1]
    ocg = w_l2g.shape[-1]
    C = cl + cg

    wc = jnp.zeros((3, 3, C, ocl + ocg), jnp.float32)
    wc = wc.at[:, :, :cl, :ocl].set(w_l2l)
    wc = wc.at[:, :, cl:, :ocl].set(w_g2l)
    wc = wc.at[:, :, :cl, ocl:].set(w_l2g)
    wc = wc.reshape(9 * C, ocl + ocg).astype(jnp.bfloat16)

    out_l, l2g = _conv3x3_dual(x_l, x_g, wc, ocl, ocg)

    s1, b1 = _bn_scale_bias(bn1_gamma, bn1_beta, bn1_mean, bn1_var)
    sfu, bfu = _bn_scale_bias(fu_bn_gamma, fu_bn_beta, fu_bn_mean, fu_bn_var)
    slf, blf = _bn_scale_bias(lfu_bn_gamma, lfu_bn_beta, lfu_bn_mean, lfu_bn_var)
    sb1 = jnp.stack([s1, b1]).astype(jnp.float32)
    sbfu = jnp.stack([sfu, bfu]).astype(jnp.float32)
    sblfu = jnp.stack([slf, blf]).astype(jnp.float32)

    y, fu, xs = _spectral_pipeline(x_g, l2g, w1, w_fu, w_lfu, w2,
                                   sb1, sbfu, sblfu)
    out_g = _conv2_fused(y.reshape(x_g.shape[:3] + (w1.shape[-1],)),
                         fu, xs, l2g, w2)
    return out_l, out_g


# speedup vs baseline: 1.5694x; 1.0038x over previous
import math

import jax
import jax.numpy as jnp
from jax import lax
from jax.experimental import pallas as pl
from jax.experimental.pallas import tpu as pltpu


def _bn_scale_bias(gamma, beta, mean, var, eps=1e-5):
    s = gamma / jnp.sqrt(var + eps)
    return s, beta - mean * s


def _make_conv_body(th, W, cl, cg, ocl):
    C = cl + cg
    bf = jnp.bfloat16

    def body(xl_ref, xg_ref, tl_ref, tg_ref, bl_ref, bg_ref, w_ref,
             outl_ref, l2g_ref, xp_ref, col_ref):
        i = pl.program_id(1)
        n = pl.num_programs(1)

        xp_ref[1:th + 1, 1:W + 1, :cl] = xl_ref[0].astype(bf)
        xp_ref[1:th + 1, 1:W + 1, cl:] = xg_ref[0].astype(bf)

        @pl.when(i == 0)
        def _():
            xp_ref[0:1, 1:W + 1, :cl] = xl_ref[0, 1:2].astype(bf)
            xp_ref[0:1, 1:W + 1, cl:] = xg_ref[0, 1:2].astype(bf)

        @pl.when(i > 0)
        def _():
            xp_ref[0:1, 1:W + 1, :cl] = tl_ref[0, 7:8].astype(bf)
            xp_ref[0:1, 1:W + 1, cl:] = tg_ref[0, 7:8].astype(bf)

        @pl.when(i == n - 1)
        def _():
            xp_ref[th + 1:th + 2, 1:W + 1, :cl] = xl_ref[0, th - 2:th - 1].astype(bf)
            xp_ref[th + 1:th + 2, 1:W + 1, cl:] = xg_ref[0, th - 2:th - 1].astype(bf)

        @pl.when(i < n - 1)
        def _():
            xp_ref[th + 1:th + 2, 1:W + 1, :cl] = bl_ref[0, 0:1].astype(bf)
            xp_ref[th + 1:th + 2, 1:W + 1, cl:] = bg_ref[0, 0:1].astype(bf)

        xp_ref[:, 0:1, :] = xp_ref[:, 2:3, :]
        xp_ref[:, W + 1:W + 2, :] = xp_ref[:, W - 1:W, :]

        for dy in range(3):
            for dx in range(3):
                t = dy * 3 + dx
                col_ref[:, t * C:(t + 1) * C] = (
                    xp_ref[dy:dy + th, dx:dx + W, :].reshape(th * W, C))

        y = jnp.dot(col_ref[...], w_ref[...],
                    preferred_element_type=jnp.float32)
        outl_ref[0] = y[:, :ocl]
        l2g_ref[0] = y[:, ocl:].astype(bf)

    return body


def _conv3x3_dual(x_l, x_g, wc, ocl, ocg, th=64):
    B, H, W, cl = x_l.shape
    cg = x_g.shape[-1]
    C = cl + cg
    n_th = H // th
    thb = th // 8

    outl, l2g = pl.pallas_call(
        _make_conv_body(th, W, cl, cg, ocl),
        out_shape=(jax.ShapeDtypeStruct((B, H * W, ocl), jnp.float32),
                   jax.ShapeDtypeStruct((B, H * W, ocg), jnp.bfloat16)),
        grid_spec=pltpu.PrefetchScalarGridSpec(
            num_scalar_prefetch=0,
            grid=(B, n_th),
            in_specs=[
                pl.BlockSpec((1, th, W, cl), lambda b, i: (b, i, 0, 0)),
                pl.BlockSpec((1, th, W, cg), lambda b, i: (b, i, 0, 0)),
                pl.BlockSpec((1, 8, W, cl),
                             lambda b, i: (b, jnp.maximum(i * thb - 1, 0), 0, 0)),
                pl.BlockSpec((1, 8, W, cg),
                             lambda b, i: (b, jnp.maximum(i * thb - 1, 0), 0, 0)),
                pl.BlockSpec((1, 8, W, cl),
                             lambda b, i: (b, jnp.minimum((i + 1) * thb,
                                                          H // 8 - 1), 0, 0)),
                pl.BlockSpec((1, 8, W, cg),
                             lambda b, i: (b, jnp.minimum((i + 1) * thb,
                                                          H // 8 - 1), 0, 0)),
                pl.BlockSpec((9 * C, ocl + ocg), lambda b, i: (0, 0)),
            ],
            out_specs=[
                pl.BlockSpec((1, th * W, ocl), lambda b, i: (b, i, 0)),
                pl.BlockSpec((1, th * W, ocg), lambda b, i: (b, i, 0)),
            ],
            scratch_shapes=[
                pltpu.VMEM((th + 2, W + 2, C), jnp.bfloat16),
                pltpu.VMEM((th * W, 9 * C), jnp.bfloat16),
            ],
        ),
        compiler_params=pltpu.CompilerParams(
            dimension_semantics=("parallel", "parallel"),
            vmem_limit_bytes=96 << 20),
    )(x_l, x_g, x_l, x_g, x_l, x_g, wc)
    return (outl.reshape(B, H, W, ocl), l2g.reshape(B, H, W, ocg))


def _pw_body(x_ref, w_ref, sb_ref, out_ref):
    y = jnp.dot(x_ref[...].astype(jnp.bfloat16), w_ref[...],
                preferred_element_type=jnp.float32)
    y = y * sb_ref[0:1] + sb_ref[1:2]
    out_ref[...] = jnp.maximum(y, 0.0)


def _pw_affine_relu(x, w, scale, bias, tm=1024):
    lead = x.shape[:-1]
    Cin = x.shape[-1]
    Cout = w.shape[-1]
    M = int(math.prod(lead))
    grid = -(-M // tm)
    sb = jnp.stack([scale, bias]).astype(jnp.float32)
    out = pl.pallas_call(
        _pw_body,
        out_shape=jax.ShapeDtypeStruct((M, Cout), jnp.float32),
        grid_spec=pltpu.PrefetchScalarGridSpec(
            num_scalar_prefetch=0,
            grid=(grid,),
            in_specs=[
                pl.BlockSpec((tm, Cin), lambda i: (i, 0)),
                pl.BlockSpec((Cin, Cout), lambda i: (0, 0)),
                pl.BlockSpec((2, Cout), lambda i: (0, 0)),
            ],
            out_specs=pl.BlockSpec((tm, Cout), lambda i: (i, 0)),
        ),
        compiler_params=pltpu.CompilerParams(
            dimension_semantics=("parallel",)),
    )(x.reshape(M, Cin), w.astype(jnp.bfloat16), sb)
    return out.reshape(lead + (Cout,))


def _conv2_body(y_ref, fu_ref, xs_ref, l2g_ref, w_ref, out_ref):
    s = (y_ref[0].astype(jnp.float32) + fu_ref[0].astype(jnp.float32)
         + xs_ref[0].astype(jnp.float32))
    Hh, Wh, c = s.shape
    o = jnp.dot(s.reshape(Hh * Wh, c).astype(jnp.bfloat16), w_ref[...],
                preferred_element_type=jnp.float32)
    o = o + l2g_ref[0].reshape(Hh * Wh, o.shape[-1])
    out_ref[0] = o.reshape(Hh, Wh, o.shape[-1])


def _conv2_fused(y, fu, xs_small, l2g, w):
    B, H, W, c = y.shape
    Cout = w.shape[-1]
    Hh, Wh = H // 2, W // 2
    out = pl.pallas_call(
        _conv2_body,
        out_shape=jax.ShapeDtypeStruct((B, H, W, Cout), jnp.float32),
        grid_spec=pltpu.PrefetchScalarGridSpec(
            num_scalar_prefetch=0,
            grid=(B, 2, 2),
            in_specs=[
                pl.BlockSpec((1, Hh, Wh, c), lambda b, i, j: (b, i, j, 0)),
                pl.BlockSpec((1, Hh, Wh, c), lambda b, i, j: (b, i, j, 0)),
                pl.BlockSpec((1, Hh, Wh, c), lambda b, i, j: (b, 0, 0, 0)),
                pl.BlockSpec((1, Hh, Wh, Cout), lambda b, i, j: (b, i, j, 0)),
                pl.BlockSpec((c, Cout), lambda b, i, j: (0, 0)),
            ],
            out_specs=pl.BlockSpec((1, Hh, Wh, Cout),
                                   lambda b, i, j: (b, i, j, 0)),
        ),
        compiler_params=pltpu.CompilerParams(
            dimension_semantics=("parallel", "parallel", "parallel")),
    )(y, fu, xs_small, l2g, w.astype(jnp.bfloat16))
    return out


def _dft_mats(Hh, Ww, Vp):
    import numpy as np
    V = Ww // 2 + 1
    u = np.arange(Hh)
    th = 2.0 * np.pi * np.outer(u, u) / Hh
    fhr = np.cos(th) / np.sqrt(Hh)
    fhi = -np.sin(th) / np.sqrt(Hh)
    w = np.arange(Ww)
    v = np.arange(Vp)
    ph = 2.0 * np.pi * np.outer(w, v) / Ww
    mask = (v < V).astype(np.float64)
    fwr = np.cos(ph) / np.sqrt(Ww) * mask
    fwi = -np.sin(ph) / np.sqrt(Ww) * mask
    ihr = np.cos(th) / np.sqrt(Hh)
    ihi = np.sin(th) / np.sqrt(Hh)
    alpha = np.where((v == 0) | (v == Ww // 2), 1.0, 2.0) * mask
    pw = 2.0 * np.pi * np.outer(v, w) / Ww
    icw = alpha[:, None] * np.cos(pw) / np.sqrt(Ww)
    isw = -alpha[:, None] * np.sin(pw) / np.sqrt(Ww)
    return [jnp.asarray(m, jnp.bfloat16)
            for m in (fhr, fhi, fwr, fwi, ihr, ihi, icw, isw)]


def _bspec(shape, blocked_dims=1):
    if blocked_dims == 0:
        return pl.BlockSpec(shape, lambda b: tuple(0 for _ in shape))
    return pl.BlockSpec((1,) + shape[1:],
                        lambda b: (b,) + tuple(0 for _ in shape[1:]))


def _batch_call(body, ins, blocked, out_shapes, out_dtype=jnp.bfloat16):
    B = ins[0].shape[0]
    return pl.pallas_call(
        body,
        out_shape=tuple(jax.ShapeDtypeStruct(s, out_dtype)
                        for s in out_shapes),
        grid_spec=pltpu.PrefetchScalarGridSpec(
            num_scalar_prefetch=0,
            grid=(B,),
            in_specs=[_bspec(a.shape, d) for a, d in zip(ins, blocked)],
            out_specs=[_bspec(s) for s in out_shapes],
        ),
        compiler_params=pltpu.CompilerParams(
            dimension_semantics=("parallel",)),
    )(*ins)


_BF = jnp.bfloat16
_F32 = jnp.float32


def _mxt(x_bf, i_ref):
    return lax.dot_general(i_ref[...], x_bf, (((1,), (1,)), ((), ())),
                           preferred_element_type=_F32)


def _make_pw2_body(HW, cg):
    def body(x_ref, w_ref, sb_ref, ic_ref, y_ref, yt_ref):
        x = x_ref[0].reshape(HW, cg).astype(_BF)
        y = jnp.dot(x, w_ref[...], preferred_element_type=_F32)
        y = jnp.maximum(y * sb_ref[0:1] + sb_ref[1:2], 0.0)
        yb = y.astype(_BF)
        y_ref[0] = yb
        yt_ref[0] = _mxt(yb, ic_ref).astype(_BF)
    return body


def _fwdw_body(yt_ref, tt_ref, fwr, fwi, gwr, gwi, iv, iv2,
               crt_ref, cit_ref, lrt_ref, lit_ref):
    yt = yt_ref[0]
    zr = jnp.dot(yt, fwr[...], preferred_element_type=_F32)
    zi = jnp.dot(yt, fwi[...], preferred_element_type=_F32)
    crt_ref[0] = _mxt(zr.astype(_BF), iv).astype(_BF)
    cit_ref[0] = _mxt(zi.astype(_BF), iv).astype(_BF)
    tt = tt_ref[0]
    wr = jnp.dot(tt, gwr[...], preferred_element_type=_F32)
    wi = jnp.dot(tt, gwi[...], preferred_element_type=_F32)
    lrt_ref[0] = _mxt(wr.astype(_BF), iv2).astype(_BF)
    lit_ref[0] = _mxt(wi.astype(_BF), iv2).astype(_BF)


def _fwdh_body(cr_ref, ci_ref, lr_ref, li_ref, fhr, fhi, ghr, ghi, iu, iu2,
               fr_ref, fi_ref, gr_ref, gi_ref):
    cr = cr_ref[0]
    ci = ci_ref[0]
    fr = (jnp.dot(cr, fhr[...], preferred_element_type=_F32)
          - jnp.dot(ci, fhi[...], preferred_element_type=_F32))
    fi = (jnp.dot(cr, fhi[...], preferred_element_type=_F32)
          + jnp.dot(ci, fhr[...], preferred_element_type=_F32))
    fr_ref[0] = _mxt(fr.astype(_BF), iu).astype(_BF)
    fi_ref[0] = _mxt(fi.astype(_BF), iu).astype(_BF)
    lr = lr_ref[0]
    li = li_ref[0]
    gr = (jnp.dot(lr, ghr[...], preferred_element_type=_F32)
          - jnp.dot(li, ghi[...], preferred_element_type=_F32))
    gi = (jnp.dot(lr, ghi[...], preferred_element_type=_F32)
          + jnp.dot(li, ghr[...], preferred_element_type=_F32))
    gr_ref[0] = _mxt(gr.astype(_BF), iu2).astype(_BF)
    gi_ref[0] = _mxt(gi.astype(_BF), iu2).astype(_BF)


def _make_freqconv_body(cc):
    def body(fr_ref, fi_ref, lr_ref, li_ref, wfu, wlfu, sbf, sbl,
             gr_ref, gi_ref, hr_ref, hi_ref):
        fr = fr_ref[0]
        fi = fi_ref[0]
        g = (jnp.dot(fr, wfu[:cc], preferred_element_type=_F32)
             + jnp.dot(fi, wfu[cc:], preferred_element_type=_F32))
        g = jnp.maximum(g * sbf[0:1] + sbf[1:2], 0.0)
        gr_ref[0] = g[:, :cc].astype(_BF)
        gi_ref[0] = g[:, cc:].astype(_BF)
        lr = lr_ref[0]
        li = li_ref[0]
        h = (jnp.dot(lr, wlfu[:cc], preferred_element_type=_F32)
             + jnp.dot(li, wlfu[cc:], preferred_element_type=_F32))
        h = jnp.maximum(h * sbl[0:1] + sbl[1:2], 0.0)
        hr_ref[0] = h[:, :cc].astype(_BF)
        hi_ref[0] = h[:, cc:].astype(_BF)
    return body


def _mxt2(x_bf, i_ref):
    return lax.dot_general(x_bf, i_ref[...], (((0,), (0,)), ((), ())),
                           preferred_element_type=_F32)


def _invh_body(gr_ref, gi_ref, hr_ref, hi_ref, ihr, ihi, jhr, jhi, iu, iu2,
               dr_ref, di_ref, er_ref, ei_ref):
    gr = gr_ref[0]
    gi = gi_ref[0]
    dr = (jnp.dot(ihr[...], gr, preferred_element_type=_F32)
          - jnp.dot(ihi[...], gi, preferred_element_type=_F32))
    di = (jnp.dot(ihr[...], gi, preferred_element_type=_F32)
          + jnp.dot(ihi[...], gr, preferred_element_type=_F32))
    dr_ref[0] = _mxt2(dr.astype(_BF), iu).astype(_BF)
    di_ref[0] = _mxt2(di.astype(_BF), iu).astype(_BF)
    hr = hr_ref[0]
    hi = hi_ref[0]
    er = (jnp.dot(jhr[...], hr, preferred_element_type=_F32)
          - jnp.dot(jhi[...], hi, preferred_element_type=_F32))
    ei = (jnp.dot(jhr[...], hi, preferred_element_type=_F32)
          + jnp.dot(jhi[...], hr, preferred_element_type=_F32))
    er_ref[0] = _mxt2(er.astype(_BF), iu2).astype(_BF)
    ei_ref[0] = _mxt2(ei.astype(_BF), iu2).astype(_BF)


def _invw_body(dr_ref, di_ref, ldr_ref, ldi_ref, icw, isw, jcw, jsw,
               iw, iw2, fu_ref, xs_ref):
    dr = dr_ref[0]
    di = di_ref[0]
    fu = (lax.dot_general(icw[...], dr, (((0,), (0,)), ((), ())),
                          preferred_element_type=_F32)
          + lax.dot_general(isw[...], di, (((0,), (0,)), ((), ())),
                            preferred_element_type=_F32))
    fu_ref[0] = _mxt2(fu.astype(_BF), iw).astype(_BF)
    ldr = ldr_ref[0]
    ldi = ldi_ref[0]
    xs = (lax.dot_general(jcw[...], ldr, (((0,), (0,)), ((), ())),
                          preferred_element_type=_F32)
          + lax.dot_general(jsw[...], ldi, (((0,), (0,)), ((), ())),
                            preferred_element_type=_F32))
    xs_ref[0] = _mxt2(xs.astype(_BF), iw2).astype(_BF)


def _spectral_pipeline(x_g, l2g, w1, w_fu, w_lfu, w2, sb1, sbfu, sblfu):
    B, H, W, cg = x_g.shape
    c = w1.shape[-1]
    H2, W2 = H // 2, W // 2
    c4 = c // 4
    Vp = ((W // 2 + 1) + 7) // 8 * 8
    Vp2 = ((W2 // 2 + 1) + 7) // 8 * 8
    fhr, fhi, fwr, fwi, ihr, ihi, icw, isw = _dft_mats(H, W, Vp)
    ghr, ghi, gwr, gwi, jhr, jhi, jcw, jsw = _dft_mats(H2, W2, Vp2)
    ic = jnp.eye(c, dtype=_BF)
    iv, iv2 = jnp.eye(Vp, dtype=_BF), jnp.eye(Vp2, dtype=_BF)
    iu, iu2 = jnp.eye(H, dtype=_BF), jnp.eye(H2, dtype=_BF)

    y, yt = _batch_call(
        _make_pw2_body(H * W, cg),
        [x_g, w1.astype(_BF), sb1, ic],
        [1, 0, 0, 0],
        [(B, H * W, c), (B, c, H * W)])

    yt4 = yt.reshape(B, c, H, W)
    tt = jnp.concatenate(
        [yt4[:, :c4, :H2, :W2], yt4[:, :c4, H2:, :W2],
         yt4[:, :c4, :H2, W2:], yt4[:, :c4, H2:, W2:]], axis=1)

    crt, cit, lrt, lit = _batch_call(
        _fwdw_body,
        [yt4.reshape(B, c * H, W), tt.reshape(B, c * H2, W2),
         fwr, fwi, gwr, gwi, iv, iv2],
        [1, 1, 0, 0, 0, 0, 0, 0],
        [(B, Vp, c * H), (B, Vp, c * H),
         (B, Vp2, c * H2), (B, Vp2, c * H2)])

    rs1 = lambda a, vp, hh: a.reshape(B, vp * c, hh)
    fr, fi, glr, gli = _batch_call(
        _fwdh_body,
        [rs1(crt, Vp, H), rs1(cit, Vp, H), rs1(lrt, Vp2, H2),
         rs1(lit, Vp2, H2), fhr, fhi, ghr, ghi, iu, iu2],
        [1, 1, 1, 1, 0, 0, 0, 0, 0, 0],
        [(B, H, Vp * c), (B, H, Vp * c),
         (B, H2, Vp2 * c), (B, H2, Vp2 * c)])

    rs2 = lambda a, hh, vp: a.reshape(B, hh * vp, c)
    gr, gi, hr, hi = _batch_call(
        _make_freqconv_body(c),
        [rs2(fr, H, Vp), rs2(fi, H, Vp), rs2(glr, H2, Vp2),
         rs2(gli, H2, Vp2), w_fu.astype(_BF), w_lfu.astype(_BF),
         sbfu, sblfu],
        [1, 1, 1, 1, 0, 0, 0, 0],
        [(B, H * Vp, c), (B, H * Vp, c),
         (B, H2 * Vp2, c), (B, H2 * Vp2, c)])

    iw, iw2 = jnp.eye(W, dtype=_BF), jnp.eye(W2, dtype=_BF)
    rs3 = lambda a, hh, vp: a.reshape(B, hh, vp * c)
    dr, di, ldr, ldi = _batch_call(
        _invh_body,
        [rs3(gr, H, Vp), rs3(gi, H, Vp), rs3(hr, H2, Vp2),
         rs3(hi, H2, Vp2), ihr, ihi, jhr, jhi, iu, iu2],
        [1, 1, 1, 1, 0, 0, 0, 0, 0, 0],
        [(B, Vp * c, H), (B, Vp * c, H),
         (B, Vp2 * c, H2), (B, Vp2 * c, H2)])

    rs4 = lambda a, vp, hh: a.reshape(B, vp, c * hh)
    fu_t, xs_t = _batch_call(
        _invw_body,
        [rs4(dr, Vp, H), rs4(di, Vp, H), rs4(ldr, Vp2, H2),
         rs4(ldi, Vp2, H2), icw, isw, jcw, jsw, iw, iw2],
        [1, 1, 1, 1, 0, 0, 0, 0, 0, 0],
        [(B, c * H, W), (B, c * H2, W2)])

    fu = jnp.transpose(fu_t.reshape(B, c, H, W), (0, 2, 3, 1))
    xs = jnp.transpose(xs_t.reshape(B, c, H2, W2), (0, 2, 3, 1))
    return y, fu, xs


def _lfu_fold(y):
    B, H, W, c = y.shape
    c4 = c // 4
    t = y[..., :c4]
    t = jnp.concatenate([t[:, : H // 2], t[:, H // 2:]], axis=-1)
    t = jnp.concatenate([t[:, :, : W // 2], t[:, :, W // 2:]], axis=-1)
    return t


def _fourier_unit(t, w, gamma, beta, mean, var):
    Hh, Ww, cch = t.shape[1], t.shape[2], t.shape[3]
    f = jnp.fft.rfft2(t, axes=(1, 2), norm="ortho")
    fr = jnp.concatenate([f.real, f.imag], axis=-1).astype(jnp.float32)
    s, b = _bn_scale_bias(gamma, beta, mean, var)
    g = _pw_affine_relu(fr, w, s, b)
    gc = lax.complex(g[..., :cch], g[..., cch:])
    return jnp.fft.irfft2(gc, s=(Hh, Ww), axes=(1, 2),
                          norm="ortho").astype(jnp.float32)


def kernel(x_l, x_g, w_l2l, w_g2l, w_l2g, w1, w_fu, w_lfu, w2,
           bn1_gamma, bn1_beta, bn1_mean, bn1_var,
           fu_bn_gamma, fu_bn_beta, fu_bn_mean, fu_bn_var,
           lfu_bn_gamma, lfu_bn_beta, lfu_bn_mean, lfu_bn_var):
    B, H, W, cl = x_l.shape
    cg = x_g.shape[-1]
    ocl = w_l2l.shape[-1]
    ocg = w_l2g.shape[-1]
    C = cl + cg

    wc = jnp.zeros((3, 3, C, ocl + ocg), jnp.float32)
    wc = wc.at[:, :, :cl, :ocl].set(w_l2l)
    wc = wc.at[:, :, cl:, :ocl].set(w_g2l)
    wc = wc.at[:, :, :cl, ocl:].set(w_l2g)
    wc = wc.reshape(9 * C, ocl + ocg).astype(jnp.bfloat16)

    out_l, l2g = _conv3x3_dual(x_l, x_g, wc, ocl, ocg)

    s1, b1 = _bn_scale_bias(bn1_gamma, bn1_beta, bn1_mean, bn1_var)
    sfu, bfu = _bn_scale_bias(fu_bn_gamma, fu_bn_beta, fu_bn_mean, fu_bn_var)
    slf, blf = _bn_scale_bias(lfu_bn_gamma, lfu_bn_beta, lfu_bn_mean, lfu_bn_var)
    sb1 = jnp.stack([s1, b1]).astype(jnp.float32)
    sbfu = jnp.stack([sfu, bfu]).astype(jnp.float32)
    sblfu = jnp.stack([slf, blf]).astype(jnp.float32)

    y, fu, xs = _spectral_pipeline(x_g, l2g, w1, w_fu, w_lfu, w2,
                                   sb1, sbfu, sblfu)
    out_g = _conv2_fused(y.reshape(x_g.shape[:3] + (w1.shape[-1],)),
                         fu, xs, l2g, w2)
    return out_l, out_g


# speedup vs baseline: 1.6213x; 1.0331x over previous
import math

import jax
import jax.numpy as jnp
from jax import lax
from jax.experimental import pallas as pl
from jax.experimental.pallas import tpu as pltpu


def _bn_scale_bias(gamma, beta, mean, var, eps=1e-5):
    s = gamma / jnp.sqrt(var + eps)
    return s, beta - mean * s


def _make_conv_body(th, W, cl, cg, ocl):
    C = cl + cg
    bf = jnp.bfloat16

    def body(xl_ref, xg_ref, tl_ref, tg_ref, bl_ref, bg_ref, w_ref,
             outl_ref, l2g_ref, xp_ref, col_ref):
        i = pl.program_id(1)
        n = pl.num_programs(1)

        xp_ref[1:th + 1, 1:W + 1, :cl] = xl_ref[0].astype(bf)
        xp_ref[1:th + 1, 1:W + 1, cl:] = xg_ref[0].astype(bf)

        @pl.when(i == 0)
        def _():
            xp_ref[0:1, 1:W + 1, :cl] = xl_ref[0, 1:2].astype(bf)
            xp_ref[0:1, 1:W + 1, cl:] = xg_ref[0, 1:2].astype(bf)

        @pl.when(i > 0)
        def _():
            xp_ref[0:1, 1:W + 1, :cl] = tl_ref[0, 7:8].astype(bf)
            xp_ref[0:1, 1:W + 1, cl:] = tg_ref[0, 7:8].astype(bf)

        @pl.when(i == n - 1)
        def _():
            xp_ref[th + 1:th + 2, 1:W + 1, :cl] = xl_ref[0, th - 2:th - 1].astype(bf)
            xp_ref[th + 1:th + 2, 1:W + 1, cl:] = xg_ref[0, th - 2:th - 1].astype(bf)

        @pl.when(i < n - 1)
        def _():
            xp_ref[th + 1:th + 2, 1:W + 1, :cl] = bl_ref[0, 0:1].astype(bf)
            xp_ref[th + 1:th + 2, 1:W + 1, cl:] = bg_ref[0, 0:1].astype(bf)

        xp_ref[:, 0:1, :] = xp_ref[:, 2:3, :]
        xp_ref[:, W + 1:W + 2, :] = xp_ref[:, W - 1:W, :]

        for dy in range(3):
            for dx in range(3):
                t = dy * 3 + dx
                col_ref[:, t * C:(t + 1) * C] = (
                    xp_ref[dy:dy + th, dx:dx + W, :].reshape(th * W, C))

        y = jnp.dot(col_ref[...], w_ref[...],
                    preferred_element_type=jnp.float32)
        outl_ref[0] = y[:, :ocl]
        l2g_ref[0] = y[:, ocl:].astype(bf)

    return body


def _conv3x3_dual(x_l, x_g, wc, ocl, ocg, th=64):
    B, H, W, cl = x_l.shape
    cg = x_g.shape[-1]
    C = cl + cg
    n_th = H // th
    thb = th // 8

    outl, l2g = pl.pallas_call(
        _make_conv_body(th, W, cl, cg, ocl),
        out_shape=(jax.ShapeDtypeStruct((B, H * W, ocl), jnp.float32),
                   jax.ShapeDtypeStruct((B, H * W, ocg), jnp.bfloat16)),
        grid_spec=pltpu.PrefetchScalarGridSpec(
            num_scalar_prefetch=0,
            grid=(B, n_th),
            in_specs=[
                pl.BlockSpec((1, th, W, cl), lambda b, i: (b, i, 0, 0)),
                pl.BlockSpec((1, th, W, cg), lambda b, i: (b, i, 0, 0)),
                pl.BlockSpec((1, 8, W, cl),
                             lambda b, i: (b, jnp.maximum(i * thb - 1, 0), 0, 0)),
                pl.BlockSpec((1, 8, W, cg),
                             lambda b, i: (b, jnp.maximum(i * thb - 1, 0), 0, 0)),
                pl.BlockSpec((1, 8, W, cl),
                             lambda b, i: (b, jnp.minimum((i + 1) * thb,
                                                          H // 8 - 1), 0, 0)),
                pl.BlockSpec((1, 8, W, cg),
                             lambda b, i: (b, jnp.minimum((i + 1) * thb,
                                                          H // 8 - 1), 0, 0)),
                pl.BlockSpec((9 * C, ocl + ocg), lambda b, i: (0, 0)),
            ],
            out_specs=[
                pl.BlockSpec((1, th * W, ocl), lambda b, i: (b, i, 0)),
                pl.BlockSpec((1, th * W, ocg), lambda b, i: (b, i, 0)),
            ],
            scratch_shapes=[
                pltpu.VMEM((th + 2, W + 2, C), jnp.bfloat16),
                pltpu.VMEM((th * W, 9 * C), jnp.bfloat16),
            ],
        ),
        compiler_params=pltpu.CompilerParams(
            dimension_semantics=("parallel", "parallel"),
            vmem_limit_bytes=96 << 20),
    )(x_l, x_g, x_l, x_g, x_l, x_g, wc)
    return (outl.reshape(B, H, W, ocl), l2g.reshape(B, H, W, ocg))


def _pw_body(x_ref, w_ref, sb_ref, out_ref):
    y = jnp.dot(x_ref[...].astype(jnp.bfloat16), w_ref[...],
                preferred_element_type=jnp.float32)
    y = y * sb_ref[0:1] + sb_ref[1:2]
    out_ref[...] = jnp.maximum(y, 0.0)


def _pw_affine_relu(x, w, scale, bias, tm=1024):
    lead = x.shape[:-1]
    Cin = x.shape[-1]
    Cout = w.shape[-1]
    M = int(math.prod(lead))
    grid = -(-M // tm)
    sb = jnp.stack([scale, bias]).astype(jnp.float32)
    out = pl.pallas_call(
        _pw_body,
        out_shape=jax.ShapeDtypeStruct((M, Cout), jnp.float32),
        grid_spec=pltpu.PrefetchScalarGridSpec(
            num_scalar_prefetch=0,
            grid=(grid,),
            in_specs=[
                pl.BlockSpec((tm, Cin), lambda i: (i, 0)),
                pl.BlockSpec((Cin, Cout), lambda i: (0, 0)),
                pl.BlockSpec((2, Cout), lambda i: (0, 0)),
            ],
            out_specs=pl.BlockSpec((tm, Cout), lambda i: (i, 0)),
        ),
        compiler_params=pltpu.CompilerParams(
            dimension_semantics=("parallel",)),
    )(x.reshape(M, Cin), w.astype(jnp.bfloat16), sb)
    return out.reshape(lead + (Cout,))


def _conv2_body(y_ref, fu_ref, xs_ref, l2g_ref, w_ref, out_ref):
    xs = xs_ref[0]
    xs = jnp.concatenate([xs, xs], axis=0)
    xs = jnp.concatenate([xs, xs], axis=1)
    s = (y_ref[0].astype(jnp.float32) + fu_ref[0].astype(jnp.float32)
         + xs.astype(jnp.float32))
    Hh, Wh, c = s.shape
    o = jnp.dot(s.reshape(Hh * Wh, c).astype(jnp.bfloat16), w_ref[...],
                preferred_element_type=jnp.float32)
    o = o + l2g_ref[0].reshape(Hh * Wh, o.shape[-1])
    out_ref[0] = o.reshape(Hh, Wh, o.shape[-1])


def _conv2_fused(y, fu, xs_small, l2g, w):
    B, H, W, c = y.shape
    Cout = w.shape[-1]
    Hh, Wh = H // 2, W // 2
    out = pl.pallas_call(
        _conv2_body,
        out_shape=jax.ShapeDtypeStruct((B, H, W, Cout), jnp.float32),
        grid_spec=pltpu.PrefetchScalarGridSpec(
            num_scalar_prefetch=0,
            grid=(B,),
            in_specs=[
                pl.BlockSpec((1, H, W, c), lambda b: (b, 0, 0, 0)),
                pl.BlockSpec((1, H, W, c), lambda b: (b, 0, 0, 0)),
                pl.BlockSpec((1, Hh, Wh, c), lambda b: (b, 0, 0, 0)),
                pl.BlockSpec((1, H, W, Cout), lambda b: (b, 0, 0, 0)),
                pl.BlockSpec((c, Cout), lambda b: (0, 0)),
            ],
            out_specs=pl.BlockSpec((1, H, W, Cout), lambda b: (b, 0, 0, 0)),
        ),
        compiler_params=pltpu.CompilerParams(
            dimension_semantics=("parallel",)),
    )(y, fu, xs_small, l2g, w.astype(jnp.bfloat16))
    return out


def _dft_mats(Hh, Ww, Vp):
    import numpy as np
    V = Ww // 2 + 1
    u = np.arange(Hh)
    th = 2.0 * np.pi * np.outer(u, u) / Hh
    fhr = np.cos(th) / np.sqrt(Hh)
    fhi = -np.sin(th) / np.sqrt(Hh)
    w = np.arange(Ww)
    v = np.arange(Vp)
    ph = 2.0 * np.pi * np.outer(w, v) / Ww
    mask = (v < V).astype(np.float64)
    fwr = np.cos(ph) / np.sqrt(Ww) * mask
    fwi = -np.sin(ph) / np.sqrt(Ww) * mask
    ihr = np.cos(th) / np.sqrt(Hh)
    ihi = np.sin(th) / np.sqrt(Hh)
    alpha = np.where((v == 0) | (v == Ww // 2), 1.0, 2.0) * mask
    pw = 2.0 * np.pi * np.outer(v, w) / Ww
    icw = alpha[:, None] * np.cos(pw) / np.sqrt(Ww)
    isw = -alpha[:, None] * np.sin(pw) / np.sqrt(Ww)
    return [jnp.asarray(m, jnp.bfloat16)
            for m in (fhr, fhi, fwr, fwi, ihr, ihi, icw, isw)]


def _bspec(shape, blocked_dims=1):
    if blocked_dims == 0:
        return pl.BlockSpec(shape, lambda b: tuple(0 for _ in shape))
    return pl.BlockSpec((1,) + shape[1:],
                        lambda b: (b,) + tuple(0 for _ in shape[1:]))


def _batch_call(body, ins, blocked, out_shapes, out_dtype=jnp.bfloat16):
    B = ins[0].shape[0]
    return pl.pallas_call(
        body,
        out_shape=tuple(jax.ShapeDtypeStruct(s, out_dtype)
                        for s in out_shapes),
        grid_spec=pltpu.PrefetchScalarGridSpec(
            num_scalar_prefetch=0,
            grid=(B,),
            in_specs=[_bspec(a.shape, d) for a, d in zip(ins, blocked)],
            out_specs=[_bspec(s) for s in out_shapes],
        ),
        compiler_params=pltpu.CompilerParams(
            dimension_semantics=("parallel",)),
    )(*ins)


_BF = jnp.bfloat16
_F32 = jnp.float32


def _mxt(x_bf, i_ref):
    return lax.dot_general(i_ref[...], x_bf, (((1,), (1,)), ((), ())),
                           preferred_element_type=_F32)


def _make_pw2_body(HW, cg):
    def body(x_ref, w_ref, sb_ref, ic_ref, y_ref, yt_ref):
        x = x_ref[0].reshape(HW, cg).astype(_BF)
        y = jnp.dot(x, w_ref[...], preferred_element_type=_F32)
        y = jnp.maximum(y * sb_ref[0:1] + sb_ref[1:2], 0.0)
        yb = y.astype(_BF)
        y_ref[0] = yb
        yt_ref[0] = _mxt(yb, ic_ref).astype(_BF)
    return body


def _fwdw_body(yt_ref, tt_ref, fwr, fwi, gwr, gwi, iv, iv2,
               crt_ref, cit_ref, lrt_ref, lit_ref):
    yt = yt_ref[0]
    zr = jnp.dot(yt, fwr[...], preferred_element_type=_F32)
    zi = jnp.dot(yt, fwi[...], preferred_element_type=_F32)
    crt_ref[0] = _mxt(zr.astype(_BF), iv).astype(_BF)
    cit_ref[0] = _mxt(zi.astype(_BF), iv).astype(_BF)
    tt = tt_ref[0]
    wr = jnp.dot(tt, gwr[...], preferred_element_type=_F32)
    wi = jnp.dot(tt, gwi[...], preferred_element_type=_F32)
    lrt_ref[0] = _mxt(wr.astype(_BF), iv2).astype(_BF)
    lit_ref[0] = _mxt(wi.astype(_BF), iv2).astype(_BF)


def _fwdh_body(cr_ref, ci_ref, lr_ref, li_ref, fhr, fhi, ghr, ghi, iu, iu2,
               fr_ref, fi_ref, gr_ref, gi_ref):
    cr = cr_ref[0]
    ci = ci_ref[0]
    fr = (jnp.dot(cr, fhr[...], preferred_element_type=_F32)
          - jnp.dot(ci, fhi[...], preferred_element_type=_F32))
    fi = (jnp.dot(cr, fhi[...], preferred_element_type=_F32)
          + jnp.dot(ci, fhr[...], preferred_element_type=_F32))
    fr_ref[0] = _mxt(fr.astype(_BF), iu).astype(_BF)
    fi_ref[0] = _mxt(fi.astype(_BF), iu).astype(_BF)
    lr = lr_ref[0]
    li = li_ref[0]
    gr = (jnp.dot(lr, ghr[...], preferred_element_type=_F32)
          - jnp.dot(li, ghi[...], preferred_element_type=_F32))
    gi = (jnp.dot(lr, ghi[...], preferred_element_type=_F32)
          + jnp.dot(li, ghr[...], preferred_element_type=_F32))
    gr_ref[0] = _mxt(gr.astype(_BF), iu2).astype(_BF)
    gi_ref[0] = _mxt(gi.astype(_BF), iu2).astype(_BF)


def _make_freqconv_body(cc):
    def body(fr_ref, fi_ref, lr_ref, li_ref, wfu, wlfu, sbf, sbl,
             gr_ref, gi_ref, hr_ref, hi_ref):
        fr = fr_ref[0]
        fi = fi_ref[0]
        g = (jnp.dot(fr, wfu[:cc], preferred_element_type=_F32)
             + jnp.dot(fi, wfu[cc:], preferred_element_type=_F32))
        g = jnp.maximum(g * sbf[0:1] + sbf[1:2], 0.0)
        gr_ref[0] = g[:, :cc].astype(_BF)
        gi_ref[0] = g[:, cc:].astype(_BF)
        lr = lr_ref[0]
        li = li_ref[0]
        h = (jnp.dot(lr, wlfu[:cc], preferred_element_type=_F32)
             + jnp.dot(li, wlfu[cc:], preferred_element_type=_F32))
        h = jnp.maximum(h * sbl[0:1] + sbl[1:2], 0.0)
        hr_ref[0] = h[:, :cc].astype(_BF)
        hi_ref[0] = h[:, cc:].astype(_BF)
    return body


def _mxt2(x_bf, i_ref):
    return lax.dot_general(x_bf, i_ref[...], (((0,), (0,)), ((), ())),
                           preferred_element_type=_F32)


def _invh_body(gr_ref, gi_ref, hr_ref, hi_ref, ihr, ihi, jhr, jhi, iu, iu2,
               dr_ref, di_ref, er_ref, ei_ref):
    gr = gr_ref[0]
    gi = gi_ref[0]
    dr = (jnp.dot(ihr[...], gr, preferred_element_type=_F32)
          - jnp.dot(ihi[...], gi, preferred_element_type=_F32))
    di = (jnp.dot(ihr[...], gi, preferred_element_type=_F32)
          + jnp.dot(ihi[...], gr, preferred_element_type=_F32))
    dr_ref[0] = _mxt2(dr.astype(_BF), iu).astype(_BF)
    di_ref[0] = _mxt2(di.astype(_BF), iu).astype(_BF)
    hr = hr_ref[0]
    hi = hi_ref[0]
    er = (jnp.dot(jhr[...], hr, preferred_element_type=_F32)
          - jnp.dot(jhi[...], hi, preferred_element_type=_F32))
    ei = (jnp.dot(jhr[...], hi, preferred_element_type=_F32)
          + jnp.dot(jhi[...], hr, preferred_element_type=_F32))
    er_ref[0] = _mxt2(er.astype(_BF), iu2).astype(_BF)
    ei_ref[0] = _mxt2(ei.astype(_BF), iu2).astype(_BF)


def _invw_body(dr_ref, di_ref, ldr_ref, ldi_ref, icw, isw, jcw, jsw,
               iw, iw2, fu_ref, xs_ref):
    dr = dr_ref[0]
    di = di_ref[0]
    fu = (lax.dot_general(icw[...], dr, (((0,), (0,)), ((), ())),
                          preferred_element_type=_F32)
          + lax.dot_general(isw[...], di, (((0,), (0,)), ((), ())),
                            preferred_element_type=_F32))
    fu_ref[0] = _mxt2(fu.astype(_BF), iw).astype(_BF)
    ldr = ldr_ref[0]
    ldi = ldi_ref[0]
    xs = (lax.dot_general(jcw[...], ldr, (((0,), (0,)), ((), ())),
                          preferred_element_type=_F32)
          + lax.dot_general(jsw[...], ldi, (((0,), (0,)), ((), ())),
                            preferred_element_type=_F32))
    xs_ref[0] = _mxt2(xs.astype(_BF), iw2).astype(_BF)


def _spectral_pipeline(x_g, l2g, w1, w_fu, w_lfu, w2, sb1, sbfu, sblfu):
    B, H, W, cg = x_g.shape
    c = w1.shape[-1]
    H2, W2 = H // 2, W // 2
    c4 = c // 4
    Vp = ((W // 2 + 1) + 7) // 8 * 8
    Vp2 = ((W2 // 2 + 1) + 7) // 8 * 8
    fhr, fhi, fwr, fwi, ihr, ihi, icw, isw = _dft_mats(H, W, Vp)
    ghr, ghi, gwr, gwi, jhr, jhi, jcw, jsw = _dft_mats(H2, W2, Vp2)
    ic = jnp.eye(c, dtype=_BF)
    iv, iv2 = jnp.eye(Vp, dtype=_BF), jnp.eye(Vp2, dtype=_BF)
    iu, iu2 = jnp.eye(H, dtype=_BF), jnp.eye(H2, dtype=_BF)

    y, yt = _batch_call(
        _make_pw2_body(H * W, cg),
        [x_g, w1.astype(_BF), sb1, ic],
        [1, 0, 0, 0],
        [(B, H * W, c), (B, c, H * W)])

    yt4 = yt.reshape(B, c, H, W)
    tt = jnp.concatenate(
        [yt4[:, :c4, :H2, :W2], yt4[:, :c4, H2:, :W2],
         yt4[:, :c4, :H2, W2:], yt4[:, :c4, H2:, W2:]], axis=1)

    crt, cit, lrt, lit = _batch_call(
        _fwdw_body,
        [yt4.reshape(B, c * H, W), tt.reshape(B, c * H2, W2),
         fwr, fwi, gwr, gwi, iv, iv2],
        [1, 1, 0, 0, 0, 0, 0, 0],
        [(B, Vp, c * H), (B, Vp, c * H),
         (B, Vp2, c * H2), (B, Vp2, c * H2)])

    rs1 = lambda a, vp, hh: a.reshape(B, vp * c, hh)
    fr, fi, glr, gli = _batch_call(
        _fwdh_body,
        [rs1(crt, Vp, H), rs1(cit, Vp, H), rs1(lrt, Vp2, H2),
         rs1(lit, Vp2, H2), fhr, fhi, ghr, ghi, iu, iu2],
        [1, 1, 1, 1, 0, 0, 0, 0, 0, 0],
        [(B, H, Vp * c), (B, H, Vp * c),
         (B, H2, Vp2 * c), (B, H2, Vp2 * c)])

    rs2 = lambda a, hh, vp: a.reshape(B, hh * vp, c)
    gr, gi, hr, hi = _batch_call(
        _make_freqconv_body(c),
        [rs2(fr, H, Vp), rs2(fi, H, Vp), rs2(glr, H2, Vp2),
         rs2(gli, H2, Vp2), w_fu.astype(_BF), w_lfu.astype(_BF),
         sbfu, sblfu],
        [1, 1, 1, 1, 0, 0, 0, 0],
        [(B, H * Vp, c), (B, H * Vp, c),
         (B, H2 * Vp2, c), (B, H2 * Vp2, c)])

    iw, iw2 = jnp.eye(W, dtype=_BF), jnp.eye(W2, dtype=_BF)
    rs3 = lambda a, hh, vp: a.reshape(B, hh, vp * c)
    dr, di, ldr, ldi = _batch_call(
        _invh_body,
        [rs3(gr, H, Vp), rs3(gi, H, Vp), rs3(hr, H2, Vp2),
         rs3(hi, H2, Vp2), ihr, ihi, jhr, jhi, iu, iu2],
        [1, 1, 1, 1, 0, 0, 0, 0, 0, 0],
        [(B, Vp * c, H), (B, Vp * c, H),
         (B, Vp2 * c, H2), (B, Vp2 * c, H2)])

    rs4 = lambda a, vp, hh: a.reshape(B, vp, c * hh)
    fu_t, xs_t = _batch_call(
        _invw_body,
        [rs4(dr, Vp, H), rs4(di, Vp, H), rs4(ldr, Vp2, H2),
         rs4(ldi, Vp2, H2), icw, isw, jcw, jsw, iw, iw2],
        [1, 1, 1, 1, 0, 0, 0, 0, 0, 0],
        [(B, c * H, W), (B, c * H2, W2)])

    fu = jnp.transpose(fu_t.reshape(B, c, H, W), (0, 2, 3, 1))
    xs = jnp.transpose(xs_t.reshape(B, c, H2, W2), (0, 2, 3, 1))
    return y, fu, xs


def _lfu_fold(y):
    B, H, W, c = y.shape
    c4 = c // 4
    t = y[..., :c4]
    t = jnp.concatenate([t[:, : H // 2], t[:, H // 2:]], axis=-1)
    t = jnp.concatenate([t[:, :, : W // 2], t[:, :, W // 2:]], axis=-1)
    return t


def _fourier_unit(t, w, gamma, beta, mean, var):
    Hh, Ww, cch = t.shape[1], t.shape[2], t.shape[3]
    f = jnp.fft.rfft2(t, axes=(1, 2), norm="ortho")
    fr = jnp.concatenate([f.real, f.imag], axis=-1).astype(jnp.float32)
    s, b = _bn_scale_bias(gamma, beta, mean, var)
    g = _pw_affine_relu(fr, w, s, b)
    gc = lax.complex(g[..., :cch], g[..., cch:])
    return jnp.fft.irfft2(gc, s=(Hh, Ww), axes=(1, 2),
                          norm="ortho").astype(jnp.float32)


def kernel(x_l, x_g, w_l2l, w_g2l, w_l2g, w1, w_fu, w_lfu, w2,
           bn1_gamma, bn1_beta, bn1_mean, bn1_var,
           fu_bn_gamma, fu_bn_beta, fu_bn_mean, fu_bn_var,
           lfu_bn_gamma, lfu_bn_beta, lfu_bn_mean, lfu_bn_var):
    B, H, W, cl = x_l.shape
    cg = x_g.shape[-1]
    ocl = w_l2l.shape[-1]
    ocg = w_l2g.shape[-1]
    C = cl + cg

    wc = jnp.zeros((3, 3, C, ocl + ocg), jnp.float32)
    wc = wc.at[:, :, :cl, :ocl].set(w_l2l)
    wc = wc.at[:, :, cl:, :ocl].set(w_g2l)
    wc = wc.at[:, :, :cl, ocl:].set(w_l2g)
    wc = wc.reshape(9 * C, ocl + ocg).astype(jnp.bfloat16)

    out_l, l2g = _conv3x3_dual(x_l, x_g, wc, ocl, ocg)

    s1, b1 = _bn_scale_bias(bn1_gamma, bn1_beta, bn1_mean, bn1_var)
    sfu, bfu = _bn_scale_bias(fu_bn_gamma, fu_bn_beta, fu_bn_mean, fu_bn_var)
    slf, blf = _bn_scale_bias(lfu_bn_gamma, lfu_bn_beta, lfu_bn_mean, lfu_bn_var)
    sb1 = jnp.stack([s1, b1]).astype(jnp.float32)
    sbfu = jnp.stack([sfu, bfu]).astype(jnp.float32)
    sblfu = jnp.stack([slf, blf]).astype(jnp.float32)

    y, fu, xs = _spectral_pipeline(x_g, l2g, w1, w_fu, w_lfu, w2,
                                   sb1, sbfu, sblfu)
    out_g = _conv2_fused(y.reshape(x_g.shape[:3] + (w1.shape[-1],)),
                         fu, xs, l2g, w2)
    return out_l, out_g


# speedup vs baseline: 1.6219x; 1.0004x over previous
import math

import jax
import jax.numpy as jnp
from jax import lax
from jax.experimental import pallas as pl
from jax.experimental.pallas import tpu as pltpu


def _bn_scale_bias(gamma, beta, mean, var, eps=1e-5):
    s = gamma / jnp.sqrt(var + eps)
    return s, beta - mean * s


def _make_conv_body(th, W, cl, cg, ocl):
    C = cl + cg
    bf = jnp.bfloat16

    def body(xl_ref, xg_ref, tl_ref, tg_ref, bl_ref, bg_ref, w_ref,
             outl_ref, l2g_ref, xp_ref, col_ref):
        i = pl.program_id(1)
        n = pl.num_programs(1)

        xp_ref[1:th + 1, 1:W + 1, :cl] = xl_ref[0].astype(bf)
        xp_ref[1:th + 1, 1:W + 1, cl:] = xg_ref[0].astype(bf)

        @pl.when(i == 0)
        def _():
            xp_ref[0:1, 1:W + 1, :cl] = xl_ref[0, 1:2].astype(bf)
            xp_ref[0:1, 1:W + 1, cl:] = xg_ref[0, 1:2].astype(bf)

        @pl.when(i > 0)
        def _():
            xp_ref[0:1, 1:W + 1, :cl] = tl_ref[0, 7:8].astype(bf)
            xp_ref[0:1, 1:W + 1, cl:] = tg_ref[0, 7:8].astype(bf)

        @pl.when(i == n - 1)
        def _():
            xp_ref[th + 1:th + 2, 1:W + 1, :cl] = xl_ref[0, th - 2:th - 1].astype(bf)
            xp_ref[th + 1:th + 2, 1:W + 1, cl:] = xg_ref[0, th - 2:th - 1].astype(bf)

        @pl.when(i < n - 1)
        def _():
            xp_ref[th + 1:th + 2, 1:W + 1, :cl] = bl_ref[0, 0:1].astype(bf)
            xp_ref[th + 1:th + 2, 1:W + 1, cl:] = bg_ref[0, 0:1].astype(bf)

        xp_ref[:, 0:1, :] = xp_ref[:, 2:3, :]
        xp_ref[:, W + 1:W + 2, :] = xp_ref[:, W - 1:W, :]

        for dy in range(3):
            for dx in range(3):
                t = dy * 3 + dx
                col_ref[:, t * C:(t + 1) * C] = (
                    xp_ref[dy:dy + th, dx:dx + W, :].reshape(th * W, C))

        y = jnp.dot(col_ref[...], w_ref[...],
                    preferred_element_type=jnp.float32)
        outl_ref[0] = y[:, :ocl]
        l2g_ref[0] = y[:, ocl:].astype(bf)

    return body


def _conv3x3_dual(x_l, x_g, wc, ocl, ocg, th=64):
    B, H, W, cl = x_l.shape
    th = min(th, H)
    cg = x_g.shape[-1]
    C = cl + cg
    n_th = H // th
    thb = th // 8

    outl, l2g = pl.pallas_call(
        _make_conv_body(th, W, cl, cg, ocl),
        out_shape=(jax.ShapeDtypeStruct((B, H * W, ocl), jnp.float32),
                   jax.ShapeDtypeStruct((B, H * W, ocg), jnp.bfloat16)),
        grid_spec=pltpu.PrefetchScalarGridSpec(
            num_scalar_prefetch=0,
            grid=(B, n_th),
            in_specs=[
                pl.BlockSpec((1, th, W, cl), lambda b, i: (b, i, 0, 0)),
                pl.BlockSpec((1, th, W, cg), lambda b, i: (b, i, 0, 0)),
                pl.BlockSpec((1, 8, W, cl),
                             lambda b, i: (b, jnp.maximum(i * thb - 1, 0), 0, 0)),
                pl.BlockSpec((1, 8, W, cg),
                             lambda b, i: (b, jnp.maximum(i * thb - 1, 0), 0, 0)),
                pl.BlockSpec((1, 8, W, cl),
                             lambda b, i: (b, jnp.minimum((i + 1) * thb,
                                                          H // 8 - 1), 0, 0)),
                pl.BlockSpec((1, 8, W, cg),
                             lambda b, i: (b, jnp.minimum((i + 1) * thb,
                                                          H // 8 - 1), 0, 0)),
                pl.BlockSpec((9 * C, ocl + ocg), lambda b, i: (0, 0)),
            ],
            out_specs=[
                pl.BlockSpec((1, th * W, ocl), lambda b, i: (b, i, 0)),
                pl.BlockSpec((1, th * W, ocg), lambda b, i: (b, i, 0)),
            ],
            scratch_shapes=[
                pltpu.VMEM((th + 2, W + 2, C), jnp.bfloat16),
                pltpu.VMEM((th * W, 9 * C), jnp.bfloat16),
            ],
        ),
        compiler_params=pltpu.CompilerParams(
            dimension_semantics=("parallel", "parallel"),
            vmem_limit_bytes=96 << 20),
    )(x_l, x_g, x_l, x_g, x_l, x_g, wc)
    return (outl.reshape(B, H, W, ocl), l2g.reshape(B, H, W, ocg))


def _conv2_body(y_ref, fu_ref, xs_ref, l2g_ref, w_ref, out_ref):
    xs = xs_ref[0]
    xs = jnp.concatenate([xs, xs], axis=0)
    xs = jnp.concatenate([xs, xs], axis=1)
    s = (y_ref[0].astype(jnp.float32) + fu_ref[0].astype(jnp.float32)
         + xs.astype(jnp.float32))
    Hh, Wh, c = s.shape
    o = jnp.dot(s.reshape(Hh * Wh, c).astype(jnp.bfloat16), w_ref[...],
                preferred_element_type=jnp.float32)
    o = o + l2g_ref[0].reshape(Hh * Wh, o.shape[-1])
    out_ref[0] = o.reshape(Hh, Wh, o.shape[-1])


def _conv2_fused(y, fu, xs_small, l2g, w):
    B, H, W, c = y.shape
    Cout = w.shape[-1]
    Hh, Wh = H // 2, W // 2
    out = pl.pallas_call(
        _conv2_body,
        out_shape=jax.ShapeDtypeStruct((B, H, W, Cout), jnp.float32),
        grid_spec=pltpu.PrefetchScalarGridSpec(
            num_scalar_prefetch=0,
            grid=(B,),
            in_specs=[
                pl.BlockSpec((1, H, W, c), lambda b: (b, 0, 0, 0)),
                pl.BlockSpec((1, H, W, c), lambda b: (b, 0, 0, 0)),
                pl.BlockSpec((1, Hh, Wh, c), lambda b: (b, 0, 0, 0)),
                pl.BlockSpec((1, H, W, Cout), lambda b: (b, 0, 0, 0)),
                pl.BlockSpec((c, Cout), lambda b: (0, 0)),
            ],
            out_specs=pl.BlockSpec((1, H, W, Cout), lambda b: (b, 0, 0, 0)),
        ),
        compiler_params=pltpu.CompilerParams(
            dimension_semantics=("parallel",)),
    )(y, fu, xs_small, l2g, w.astype(jnp.bfloat16))
    return out


def _dft_mats(Hh, Ww, Vp):
    import numpy as np
    V = Ww // 2 + 1
    u = np.arange(Hh)
    th = 2.0 * np.pi * np.outer(u, u) / Hh
    fhr = np.cos(th) / np.sqrt(Hh)
    fhi = -np.sin(th) / np.sqrt(Hh)
    w = np.arange(Ww)
    v = np.arange(Vp)
    ph = 2.0 * np.pi * np.outer(w, v) / Ww
    mask = (v < V).astype(np.float64)
    fwr = np.cos(ph) / np.sqrt(Ww) * mask
    fwi = -np.sin(ph) / np.sqrt(Ww) * mask
    ihr = np.cos(th) / np.sqrt(Hh)
    ihi = np.sin(th) / np.sqrt(Hh)
    alpha = np.where((v == 0) | (v == Ww // 2), 1.0, 2.0) * mask
    pw = 2.0 * np.pi * np.outer(v, w) / Ww
    icw = alpha[:, None] * np.cos(pw) / np.sqrt(Ww)
    isw = -alpha[:, None] * np.sin(pw) / np.sqrt(Ww)
    return [jnp.asarray(m, jnp.bfloat16)
            for m in (fhr, fhi, fwr, fwi, ihr, ihi, icw, isw)]


def _bspec(shape, blocked_dims=1):
    if blocked_dims == 0:
        return pl.BlockSpec(shape, lambda b: tuple(0 for _ in shape))
    return pl.BlockSpec((1,) + shape[1:],
                        lambda b: (b,) + tuple(0 for _ in shape[1:]))


def _batch_call(body, ins, blocked, out_shapes, out_dtype=jnp.bfloat16):
    B = ins[0].shape[0]
    return pl.pallas_call(
        body,
        out_shape=tuple(jax.ShapeDtypeStruct(s, out_dtype)
                        for s in out_shapes),
        grid_spec=pltpu.PrefetchScalarGridSpec(
            num_scalar_prefetch=0,
            grid=(B,),
            in_specs=[_bspec(a.shape, d) for a, d in zip(ins, blocked)],
            out_specs=[_bspec(s) for s in out_shapes],
        ),
        compiler_params=pltpu.CompilerParams(
            dimension_semantics=("parallel",)),
    )(*ins)


_BF = jnp.bfloat16
_F32 = jnp.float32


def _mxt(x_bf, i_ref):
    return lax.dot_general(i_ref[...], x_bf, (((1,), (1,)), ((), ())),
                           preferred_element_type=_F32)


def _make_pw2_body(HW, cg):
    def body(x_ref, w_ref, sb_ref, ic_ref, y_ref, yt_ref):
        x = x_ref[0].reshape(HW, cg).astype(_BF)
        y = jnp.dot(x, w_ref[...], preferred_element_type=_F32)
        y = jnp.maximum(y * sb_ref[0:1] + sb_ref[1:2], 0.0)
        yb = y.astype(_BF)
        y_ref[0] = yb
        yt_ref[0] = _mxt(yb, ic_ref).astype(_BF)
    return body


def _fwdw_body(yt_ref, tt_ref, fwr, fwi, gwr, gwi, iv, iv2,
               crt_ref, cit_ref, lrt_ref, lit_ref):
    yt = yt_ref[0]
    zr = jnp.dot(yt, fwr[...], preferred_element_type=_F32)
    zi = jnp.dot(yt, fwi[...], preferred_element_type=_F32)
    crt_ref[0] = _mxt(zr.astype(_BF), iv).astype(_BF)
    cit_ref[0] = _mxt(zi.astype(_BF), iv).astype(_BF)
    tt = tt_ref[0]
    wr = jnp.dot(tt, gwr[...], preferred_element_type=_F32)
    wi = jnp.dot(tt, gwi[...], preferred_element_type=_F32)
    lrt_ref[0] = _mxt(wr.astype(_BF), iv2).astype(_BF)
    lit_ref[0] = _mxt(wi.astype(_BF), iv2).astype(_BF)


def _fwdh_body(cr_ref, ci_ref, lr_ref, li_ref, fhr, fhi, ghr, ghi, iu, iu2,
               fr_ref, fi_ref, gr_ref, gi_ref):
    cr = cr_ref[0]
    ci = ci_ref[0]
    fr = (jnp.dot(cr, fhr[...], preferred_element_type=_F32)
          - jnp.dot(ci, fhi[...], preferred_element_type=_F32))
    fi = (jnp.dot(cr, fhi[...], preferred_element_type=_F32)
          + jnp.dot(ci, fhr[...], preferred_element_type=_F32))
    fr_ref[0] = _mxt(fr.astype(_BF), iu).astype(_BF)
    fi_ref[0] = _mxt(fi.astype(_BF), iu).astype(_BF)
    lr = lr_ref[0]
    li = li_ref[0]
    gr = (jnp.dot(lr, ghr[...], preferred_element_type=_F32)
          - jnp.dot(li, ghi[...], preferred_element_type=_F32))
    gi = (jnp.dot(lr, ghi[...], preferred_element_type=_F32)
          + jnp.dot(li, ghr[...], preferred_element_type=_F32))
    gr_ref[0] = _mxt(gr.astype(_BF), iu2).astype(_BF)
    gi_ref[0] = _mxt(gi.astype(_BF), iu2).astype(_BF)


def _make_freqconv_body(cc):
    def body(fr_ref, fi_ref, lr_ref, li_ref, wfu, wlfu, sbf, sbl,
             gr_ref, gi_ref, hr_ref, hi_ref):
        fr = fr_ref[0]
        fi = fi_ref[0]
        g = (jnp.dot(fr, wfu[:cc], preferred_element_type=_F32)
             + jnp.dot(fi, wfu[cc:], preferred_element_type=_F32))
        g = jnp.maximum(g * sbf[0:1] + sbf[1:2], 0.0)
        gr_ref[0] = g[:, :cc].astype(_BF)
        gi_ref[0] = g[:, cc:].astype(_BF)
        lr = lr_ref[0]
        li = li_ref[0]
        h = (jnp.dot(lr, wlfu[:cc], preferred_element_type=_F32)
             + jnp.dot(li, wlfu[cc:], preferred_element_type=_F32))
        h = jnp.maximum(h * sbl[0:1] + sbl[1:2], 0.0)
        hr_ref[0] = h[:, :cc].astype(_BF)
        hi_ref[0] = h[:, cc:].astype(_BF)
    return body


def _mxt2(x_bf, i_ref):
    return lax.dot_general(x_bf, i_ref[...], (((0,), (0,)), ((), ())),
                           preferred_element_type=_F32)


def _invh_body(gr_ref, gi_ref, hr_ref, hi_ref, ihr, ihi, jhr, jhi, iu, iu2,
               dr_ref, di_ref, er_ref, ei_ref):
    gr = gr_ref[0]
    gi = gi_ref[0]
    dr = (jnp.dot(ihr[...], gr, preferred_element_type=_F32)
          - jnp.dot(ihi[...], gi, preferred_element_type=_F32))
    di = (jnp.dot(ihr[...], gi, preferred_element_type=_F32)
          + jnp.dot(ihi[...], gr, preferred_element_type=_F32))
    dr_ref[0] = _mxt2(dr.astype(_BF), iu).astype(_BF)
    di_ref[0] = _mxt2(di.astype(_BF), iu).astype(_BF)
    hr = hr_ref[0]
    hi = hi_ref[0]
    er = (jnp.dot(jhr[...], hr, preferred_element_type=_F32)
          - jnp.dot(jhi[...], hi, preferred_element_type=_F32))
    ei = (jnp.dot(jhr[...], hi, preferred_element_type=_F32)
          + jnp.dot(jhi[...], hr, preferred_element_type=_F32))
    er_ref[0] = _mxt2(er.astype(_BF), iu2).astype(_BF)
    ei_ref[0] = _mxt2(ei.astype(_BF), iu2).astype(_BF)


def _invw_body(dr_ref, di_ref, ldr_ref, ldi_ref, icw, isw, jcw, jsw,
               iw, iw2, fu_ref, xs_ref):
    dr = dr_ref[0]
    di = di_ref[0]
    fu = (lax.dot_general(icw[...], dr, (((0,), (0,)), ((), ())),
                          preferred_element_type=_F32)
          + lax.dot_general(isw[...], di, (((0,), (0,)), ((), ())),
                            preferred_element_type=_F32))
    fu_ref[0] = _mxt2(fu.astype(_BF), iw).astype(_BF)
    ldr = ldr_ref[0]
    ldi = ldi_ref[0]
    xs = (lax.dot_general(jcw[...], ldr, (((0,), (0,)), ((), ())),
                          preferred_element_type=_F32)
          + lax.dot_general(jsw[...], ldi, (((0,), (0,)), ((), ())),
                            preferred_element_type=_F32))
    xs_ref[0] = _mxt2(xs.astype(_BF), iw2).astype(_BF)


def _spectral_pipeline(x_g, l2g, w1, w_fu, w_lfu, w2, sb1, sbfu, sblfu):
    B, H, W, cg = x_g.shape
    c = w1.shape[-1]
    H2, W2 = H // 2, W // 2
    c4 = c // 4
    Vp = ((W // 2 + 1) + 7) // 8 * 8
    Vp2 = ((W2 // 2 + 1) + 7) // 8 * 8
    fhr, fhi, fwr, fwi, ihr, ihi, icw, isw = _dft_mats(H, W, Vp)
    ghr, ghi, gwr, gwi, jhr, jhi, jcw, jsw = _dft_mats(H2, W2, Vp2)
    ic = jnp.eye(c, dtype=_BF)
    iv, iv2 = jnp.eye(Vp, dtype=_BF), jnp.eye(Vp2, dtype=_BF)
    iu, iu2 = jnp.eye(H, dtype=_BF), jnp.eye(H2, dtype=_BF)

    y, yt = _batch_call(
        _make_pw2_body(H * W, cg),
        [x_g, w1.astype(_BF), sb1, ic],
        [1, 0, 0, 0],
        [(B, H * W, c), (B, c, H * W)])

    yt4 = yt.reshape(B, c, H, W)
    tt = jnp.concatenate(
        [yt4[:, :c4, :H2, :W2], yt4[:, :c4, H2:, :W2],
         yt4[:, :c4, :H2, W2:], yt4[:, :c4, H2:, W2:]], axis=1)

    crt, cit, lrt, lit = _batch_call(
        _fwdw_body,
        [yt4.reshape(B, c * H, W), tt.reshape(B, c * H2, W2),
         fwr, fwi, gwr, gwi, iv, iv2],
        [1, 1, 0, 0, 0, 0, 0, 0],
        [(B, Vp, c * H), (B, Vp, c * H),
         (B, Vp2, c * H2), (B, Vp2, c * H2)])

    rs1 = lambda a, vp, hh: a.reshape(B, vp * c, hh)
    fr, fi, glr, gli = _batch_call(
        _fwdh_body,
        [rs1(crt, Vp, H), rs1(cit, Vp, H), rs1(lrt, Vp2, H2),
         rs1(lit, Vp2, H2), fhr, fhi, ghr, ghi, iu, iu2],
        [1, 1, 1, 1, 0, 0, 0, 0, 0, 0],
        [(B, H, Vp * c), (B, H, Vp * c),
         (B, H2, Vp2 * c), (B, H2, Vp2 * c)])

    rs2 = lambda a, hh, vp: a.reshape(B, hh * vp, c)
    gr, gi, hr, hi = _batch_call(
        _make_freqconv_body(c),
        [rs2(fr, H, Vp), rs2(fi, H, Vp), rs2(glr, H2, Vp2),
         rs2(gli, H2, Vp2), w_fu.astype(_BF), w_lfu.astype(_BF),
         sbfu, sblfu],
        [1, 1, 1, 1, 0, 0, 0, 0],
        [(B, H * Vp, c), (B, H * Vp, c),
         (B, H2 * Vp2, c), (B, H2 * Vp2, c)])

    iw, iw2 = jnp.eye(W, dtype=_BF), jnp.eye(W2, dtype=_BF)
    rs3 = lambda a, hh, vp: a.reshape(B, hh, vp * c)
    dr, di, ldr, ldi = _batch_call(
        _invh_body,
        [rs3(gr, H, Vp), rs3(gi, H, Vp), rs3(hr, H2, Vp2),
         rs3(hi, H2, Vp2), ihr, ihi, jhr, jhi, iu, iu2],
        [1, 1, 1, 1, 0, 0, 0, 0, 0, 0],
        [(B, Vp * c, H), (B, Vp * c, H),
         (B, Vp2 * c, H2), (B, Vp2 * c, H2)])

    rs4 = lambda a, vp, hh: a.reshape(B, vp, c * hh)
    fu_t, xs_t = _batch_call(
        _invw_body,
        [rs4(dr, Vp, H), rs4(di, Vp, H), rs4(ldr, Vp2, H2),
         rs4(ldi, Vp2, H2), icw, isw, jcw, jsw, iw, iw2],
        [1, 1, 1, 1, 0, 0, 0, 0, 0, 0],
        [(B, c * H, W), (B, c * H2, W2)])

    fu = jnp.transpose(fu_t.reshape(B, c, H, W), (0, 2, 3, 1))
    xs = jnp.transpose(xs_t.reshape(B, c, H2, W2), (0, 2, 3, 1))
    return y, fu, xs


def kernel(x_l, x_g, w_l2l, w_g2l, w_l2g, w1, w_fu, w_lfu, w2,
           bn1_gamma, bn1_beta, bn1_mean, bn1_var,
           fu_bn_gamma, fu_bn_beta, fu_bn_mean, fu_bn_var,
           lfu_bn_gamma, lfu_bn_beta, lfu_bn_mean, lfu_bn_var):
    B, H, W, cl = x_l.shape
    cg = x_g.shape[-1]
    ocl = w_l2l.shape[-1]
    ocg = w_l2g.shape[-1]
    C = cl + cg

    wc = jnp.zeros((3, 3, C, ocl + ocg), jnp.float32)
    wc = wc.at[:, :, :cl, :ocl].set(w_l2l)
    wc = wc.at[:, :, cl:, :ocl].set(w_g2l)
    wc = wc.at[:, :, :cl, ocl:].set(w_l2g)
    wc = wc.reshape(9 * C, ocl + ocg).astype(jnp.bfloat16)

    out_l, l2g = _conv3x3_dual(x_l, x_g, wc, ocl, ocg)

    s1, b1 = _bn_scale_bias(bn1_gamma, bn1_beta, bn1_mean, bn1_var)
    sfu, bfu = _bn_scale_bias(fu_bn_gamma, fu_bn_beta, fu_bn_mean, fu_bn_var)
    slf, blf = _bn_scale_bias(lfu_bn_gamma, lfu_bn_beta, lfu_bn_mean, lfu_bn_var)
    sb1 = jnp.stack([s1, b1]).astype(jnp.float32)
    sbfu = jnp.stack([sfu, bfu]).astype(jnp.float32)
    sblfu = jnp.stack([slf, blf]).astype(jnp.float32)

    y, fu, xs = _spectral_pipeline(x_g, l2g, w1, w_fu, w_lfu, w2,
                                   sb1, sbfu, sblfu)
    out_g = _conv2_fused(y.reshape(x_g.shape[:3] + (w1.shape[-1],)),
                         fu, xs, l2g, w2)
    return out_l, out_g
